# Initial kernel scaffold; baseline (speedup 1.0000x reference)
import jax
import jax.numpy as jnp
from jax.experimental import pallas as pl


def kernel(context_entities, context_words, context_tokens, llm_compressed_tokens, edge_index, edge_type, word_edge_index, rgcn_basis, rgcn_comp, rgcn_root, rgcn_bias, word_embedding, gcn_weight, gcn_bias, ent_attn_Wa, ent_attn_b, word_attn_Wa, word_attn_b, gate_W, gate_b, rec_bias):
    raise NotImplementedError("write your pallas kernel here")


# speedup vs baseline: 1.3591x; 1.3591x over previous
import functools

import jax
import jax.numpy as jnp
from jax import lax
from jax.experimental import pallas as pl
from jax.experimental.pallas import tpu as pltpu
from jax.experimental.pallas import tpu_sc as plsc

N_ENTITY = 10000
N_WORDS = 20000
N_REL = 12
N_BASES = 8
D = 128
B = 1024
LE = 50
LW = 200
E_ENT = 320000
E_WORD = 320000

NC = 2
NS = 16
NW = NC * NS

_MESH = plsc.VectorSubcoreMesh(core_axis_name="c", subcore_axis_name="s")

CNT_Q = 7552
CNT_PAD = NS * CNT_Q
DEG_Q = 1280
DEG_PAD = NS * DEG_Q

CHUNK = 128


def _fill16(ref, val, n):
    @pl.loop(0, n, step=16)
    def _(i):
        ref[pl.ds(i, 16)] = jnp.full((16,), val, ref.dtype)


def _hist_kernel(seg_hbm, dstw_hbm, cnt_out, deg_out,
                 cnt_sp, deg_sp, zbuf, ones_v, idx_v):
    c = lax.axis_index("c")
    s = lax.axis_index("s")
    wid = c * NS + s

    _fill16(zbuf, 0.0, CNT_Q)
    pltpu.sync_copy(zbuf.at[pl.ds(0, CNT_Q)], cnt_sp.at[pl.ds(s * CNT_Q, CNT_Q)])
    pltpu.sync_copy(zbuf.at[pl.ds(0, DEG_Q)], deg_sp.at[pl.ds(s * DEG_Q, DEG_Q)])
    _fill16(ones_v, 1.0, CHUNK)
    plsc.subcore_barrier()

    n_chunks_ent = E_ENT // CHUNK
    n_chunks_word = E_WORD // CHUNK

    @pl.loop(wid, n_chunks_ent, step=NW)
    def _(j):
        pltpu.sync_copy(seg_hbm.at[pl.ds(j * CHUNK, CHUNK)], idx_v.at[0])
        pltpu.sync_copy(ones_v, cnt_sp.at[idx_v.at[0]], add=True)

    @pl.loop(wid, n_chunks_word, step=NW)
    def _(j):
        pltpu.sync_copy(dstw_hbm.at[pl.ds(j * CHUNK, CHUNK)], idx_v.at[0])
        pltpu.sync_copy(ones_v, deg_sp.at[idx_v.at[0]], add=True)

    plsc.subcore_barrier()

    pltpu.sync_copy(cnt_sp.at[pl.ds(s * CNT_Q, CNT_Q)],
                    cnt_out.at[c, pl.ds(s * CNT_Q, CNT_Q)])
    pltpu.sync_copy(deg_sp.at[pl.ds(s * DEG_Q, DEG_Q)],
                    deg_out.at[c, pl.ds(s * DEG_Q, DEG_Q)])


def _sc_histograms(seg_ent, dst_word):
    k = pl.kernel(
        _hist_kernel,
        out_type=(jax.ShapeDtypeStruct((NC, CNT_PAD), jnp.float32),
                  jax.ShapeDtypeStruct((NC, DEG_PAD), jnp.float32)),
        mesh=_MESH,
        scratch_types=[
            pltpu.VMEM_SHARED((CNT_PAD,), jnp.float32),
            pltpu.VMEM_SHARED((DEG_PAD,), jnp.float32),
            pltpu.VMEM((CNT_Q,), jnp.float32),
            pltpu.VMEM((CHUNK,), jnp.float32),
            pltpu.VMEM((1, CHUNK), jnp.int32),
        ],
    )
    cnt2, deg2 = k(seg_ent, dst_word)
    cnt = (cnt2[0] + cnt2[1])[:N_ENTITY * N_REL]
    deg = (deg2[0] + deg2[1])[:N_WORDS]
    return cnt, deg


def kernel(context_entities, context_words, context_tokens,
           llm_compressed_tokens, edge_index, edge_type, word_edge_index,
           rgcn_basis, rgcn_comp, rgcn_root, rgcn_bias,
           word_embedding, gcn_weight, gcn_bias,
           ent_attn_Wa, ent_attn_b, word_attn_Wa, word_attn_b,
           gate_W, gate_b, rec_bias):
    src, dst = edge_index[0], edge_index[1]
    rel = edge_type
    seg = dst * N_REL + rel
    gidx = rel * N_ENTITY + src
    ws, wd = word_edge_index[0], word_edge_index[1]

    cnt, deg = _sc_histograms(seg.astype(jnp.int32), wd.astype(jnp.int32))
    deg = deg + 1.0

    weight = jnp.einsum("rb,bnd->rnd", rgcn_comp, rgcn_basis).reshape(
        N_REL * N_ENTITY, D)
    winv = 1.0 / jnp.maximum(cnt, 1.0)
    msg = weight[gidx] * winv[seg][:, None]
    accE = jax.ops.segment_sum(msg, dst, num_segments=N_ENTITY)
    E = accE + rgcn_root + rgcn_bias

    dinv = 1.0 / jnp.sqrt(jnp.maximum(deg, 1.0))
    xw = word_embedding @ gcn_weight
    y = dinv[:, None] * xw
    accW = jax.ops.segment_sum(y[ws], wd, num_segments=N_WORDS)
    wrep = dinv[:, None] * (accW + y) + gcn_bias

    def attn(table, Wa, b, idx):
        tvec = jnp.tanh(table @ Wa) @ b
        sc = tvec[idx]
        mask = idx != 0
        sc = jnp.where(mask, sc, -1e30)
        a = jax.nn.softmax(sc, axis=-1)
        return jnp.einsum("bl,bld->bd", a, table[idx])

    kg = attn(E, ent_attn_Wa, ent_attn_b, context_entities)
    wa = attn(wrep, word_attn_Wa, word_attn_b, context_words)
    g = jax.nn.sigmoid(jnp.concatenate([kg, wa], -1) @ gate_W + gate_b)
    u = g * kg + (1 - g) * wa
    return u @ E.T + rec_bias


# speedup vs baseline: 2.4307x; 1.7885x over previous
import dataclasses
import functools

import jax
import jax.numpy as jnp
from jax import lax
from jax.experimental import pallas as pl
from jax.experimental.pallas import tpu as pltpu
from jax.experimental.pallas import tpu_sc as plsc

N_ENTITY = 10000
N_WORDS = 20000
N_REL = 12
N_BASES = 8
D = 128
B = 1024
LE = 50
LW = 200
E_ENT = 320000
E_WORD = 320000

NC = 2
NS = 16
NW = NC * NS

_MESH = plsc.VectorSubcoreMesh(core_axis_name="c", subcore_axis_name="s")

_SC_PARAMS = pltpu.CompilerParams()
if "needs_layout_passes" in pltpu.CompilerParams.__dataclass_fields__:
    _SC_PARAMS = dataclasses.replace(_SC_PARAMS, needs_layout_passes=False)

CNT_Q = 7552
CNT_PAD = NS * CNT_Q
DEG_Q = 1280
DEG_PAD = NS * DEG_Q

CHUNK = 128


def _fill16(ref, val, n):
    @pl.loop(0, n, step=16)
    def _(i):
        ref[pl.ds(i, 16)] = jnp.full((16,), val, ref.dtype)


def _hist_kernel(seg_hbm, dstw_hbm, cnt_out, deg_out,
                 cnt_sp, deg_sp, zbuf, ones_v, idx_v):
    c = lax.axis_index("c")
    s = lax.axis_index("s")
    wid = c * NS + s

    _fill16(zbuf, 0.0, CNT_Q)
    pltpu.sync_copy(zbuf.at[pl.ds(0, CNT_Q)], cnt_sp.at[pl.ds(s * CNT_Q, CNT_Q)])
    pltpu.sync_copy(zbuf.at[pl.ds(0, DEG_Q)], deg_sp.at[pl.ds(s * DEG_Q, DEG_Q)])
    _fill16(ones_v, 1.0, CHUNK)
    plsc.subcore_barrier()

    n_chunks_ent = E_ENT // CHUNK
    n_chunks_word = E_WORD // CHUNK

    @pl.loop(wid, n_chunks_ent, step=NW)
    def _(j):
        pltpu.sync_copy(seg_hbm.at[pl.ds(j * CHUNK, CHUNK)], idx_v.at[0])
        pltpu.sync_copy(ones_v, cnt_sp.at[idx_v.at[0]], add=True)

    @pl.loop(wid, n_chunks_word, step=NW)
    def _(j):
        pltpu.sync_copy(dstw_hbm.at[pl.ds(j * CHUNK, CHUNK)], idx_v.at[0])
        pltpu.sync_copy(ones_v, deg_sp.at[idx_v.at[0]], add=True)

    plsc.subcore_barrier()

    pltpu.sync_copy(cnt_sp.at[pl.ds(s * CNT_Q, CNT_Q)],
                    cnt_out.at[c, pl.ds(s * CNT_Q, CNT_Q)])
    pltpu.sync_copy(deg_sp.at[pl.ds(s * DEG_Q, DEG_Q)],
                    deg_out.at[c, pl.ds(s * DEG_Q, DEG_Q)])


def _sc_histograms(seg_ent, dst_word):
    k = pl.kernel(
        _hist_kernel,
        out_type=(jax.ShapeDtypeStruct((NC, CNT_PAD), jnp.float32),
                  jax.ShapeDtypeStruct((NC, DEG_PAD), jnp.float32)),
        mesh=_MESH,
        scratch_types=[
            pltpu.VMEM_SHARED((CNT_PAD,), jnp.float32),
            pltpu.VMEM_SHARED((DEG_PAD,), jnp.float32),
            pltpu.VMEM((CNT_Q,), jnp.float32),
            pltpu.VMEM((CHUNK,), jnp.float32),
            pltpu.VMEM((1, CHUNK), jnp.int32),
        ],
    )
    cnt2, deg2 = k(seg_ent, dst_word)
    cnt = (cnt2[0] + cnt2[1])[:N_ENTITY * N_REL]
    deg = (deg2[0] + deg2[1])[:N_WORDS]
    return cnt, deg


ACC_E_Q = 640
ACC_E_PAD = NS * ACC_E_Q
ACC_W_Q = 640
ACC_W_PAD = NS * ACC_W_Q


def _zero_shared_rows(acc_sp, rows, s, quota):
    @pl.loop(0, CHUNK)
    def _(r):
        @pl.loop(0, D, step=16)
        def _(k):
            rows[r, pl.ds(k, 16)] = jnp.zeros((16,), jnp.float32)

    @pl.loop(0, quota, step=CHUNK)
    def _(r):
        pltpu.sync_copy(rows, acc_sp.at[pl.ds(s * quota + r, CHUNK)])


def _ent_edge_kernel(wflat_hbm, winv_hbm, gidx_hbm, seg_hbm, dst_hbm, acc_out,
                     acc_sp, rows, gidx_v, seg_v, dst_v, w_v, sem_r, sem_w):
    c = lax.axis_index("c")
    s = lax.axis_index("s")
    wid = c * NS + s

    _zero_shared_rows(acc_sp, rows, s, ACC_E_Q)
    plsc.subcore_barrier()

    n_chunks = E_ENT // CHUNK

    @pl.loop(wid, n_chunks, step=NW)
    def _(j):
        base = j * CHUNK
        pltpu.sync_copy(gidx_hbm.at[pl.ds(base, CHUNK)], gidx_v.at[0])
        pltpu.sync_copy(seg_hbm.at[pl.ds(base, CHUNK)], seg_v.at[0])
        pltpu.sync_copy(dst_hbm.at[pl.ds(base, CHUNK)], dst_v.at[0])
        cp_r = pltpu.async_copy(wflat_hbm.at[gidx_v.at[0]], rows, sem_r)
        cp_w = pltpu.async_copy(winv_hbm.at[seg_v.at[0]], w_v, sem_w)
        cp_w.wait()
        cp_r.wait()

        @pl.loop(0, CHUNK)
        def _(e):
            wsplat = plsc.load_gather(w_v, [jnp.full((16,), e, jnp.int32)])
            for k in range(D // 16):
                sl = rows.at[e, pl.ds(k * 16, 16)]
                sl[...] = sl[...] * wsplat

        pltpu.sync_copy(rows, acc_sp.at[dst_v.at[0]], add=True)

    plsc.subcore_barrier()

    @pl.loop(0, ACC_E_Q, step=CHUNK)
    def _(r):
        pltpu.sync_copy(acc_sp.at[pl.ds(s * ACC_E_Q + r, CHUNK)],
                        acc_out.at[c, pl.ds(s * ACC_E_Q + r, CHUNK)])


def _word_edge_kernel(y_hbm, src_hbm, dst_hbm, acc_out,
                      acc_sp, rows, src_v, dl_v, lidx_v, sem_r):
    c = lax.axis_index("c")
    s = lax.axis_index("s")

    _zero_shared_rows(acc_sp, rows, s, ACC_W_Q)
    plsc.subcore_barrier()

    n_chunks = E_WORD // CHUNK
    lo = c * N_ENTITY

    @pl.loop(s, n_chunks, step=NS)
    def _(j):
        base = j * CHUNK
        pltpu.sync_copy(src_hbm.at[pl.ds(base, CHUNK)], src_v.at[0])
        pltpu.sync_copy(dst_hbm.at[pl.ds(base, CHUNK)], dl_v)
        cp_r = pltpu.async_copy(y_hbm.at[src_v.at[0]], rows, sem_r)
        for k in range(CHUNK // 16):
            v = dl_v[pl.ds(k * 16, 16)] - lo
            inb = (v >= 0) & (v < 10000)
            trash = jnp.full((16,), 10000 + k * 16, jnp.int32) + lax.iota(
                jnp.int32, 16)
            lidx_v[pl.ds(k * 16, 16)] = jnp.where(inb, v, trash)
        cp_r.wait()
        pltpu.sync_copy(rows, acc_sp.at[lidx_v], add=True)

    plsc.subcore_barrier()

    @pl.loop(0, ACC_W_Q, step=CHUNK)
    def _(r):
        pltpu.sync_copy(acc_sp.at[pl.ds(s * ACC_W_Q + r, CHUNK)],
                        acc_out.at[c, pl.ds(s * ACC_W_Q + r, CHUNK)])


def _sc_ent_edges(wflat, winv, gidx, seg, dst):
    k = pl.kernel(
        _ent_edge_kernel,
        out_type=jax.ShapeDtypeStruct((NC, ACC_E_PAD, D), jnp.float32),
        mesh=_MESH,
        scratch_types=[
            pltpu.VMEM_SHARED((ACC_E_PAD, D), jnp.float32),
            pltpu.VMEM((CHUNK, D), jnp.float32),
            pltpu.VMEM((1, CHUNK), jnp.int32),
            pltpu.VMEM((1, CHUNK), jnp.int32),
            pltpu.VMEM((1, CHUNK), jnp.int32),
            pltpu.VMEM((CHUNK,), jnp.float32),
            pltpu.SemaphoreType.DMA,
            pltpu.SemaphoreType.DMA,
        ],
        compiler_params=_SC_PARAMS,
    )
    acc2 = k(wflat, winv, gidx, seg, dst)
    return (acc2[0] + acc2[1])[:N_ENTITY]


def _sc_word_edges(y, src_w, dst_w):
    k = pl.kernel(
        _word_edge_kernel,
        out_type=jax.ShapeDtypeStruct((NC, ACC_W_PAD, D), jnp.float32),
        mesh=_MESH,
        scratch_types=[
            pltpu.VMEM_SHARED((ACC_W_PAD, D), jnp.float32),
            pltpu.VMEM((CHUNK, D), jnp.float32),
            pltpu.VMEM((1, CHUNK), jnp.int32),
            pltpu.VMEM((CHUNK,), jnp.int32),
            pltpu.VMEM((CHUNK,), jnp.int32),
            pltpu.SemaphoreType.DMA,
        ],
        compiler_params=_SC_PARAMS,
    )
    acc2 = k(y, src_w, dst_w)
    return jnp.concatenate([acc2[0, :N_ENTITY], acc2[1, :N_ENTITY]], axis=0)


def kernel(context_entities, context_words, context_tokens,
           llm_compressed_tokens, edge_index, edge_type, word_edge_index,
           rgcn_basis, rgcn_comp, rgcn_root, rgcn_bias,
           word_embedding, gcn_weight, gcn_bias,
           ent_attn_Wa, ent_attn_b, word_attn_Wa, word_attn_b,
           gate_W, gate_b, rec_bias):
    src, dst = edge_index[0], edge_index[1]
    rel = edge_type
    seg = dst * N_REL + rel
    gidx = rel * N_ENTITY + src
    ws, wd = word_edge_index[0], word_edge_index[1]

    cnt, deg = _sc_histograms(seg.astype(jnp.int32), wd.astype(jnp.int32))
    deg = deg + 1.0

    weight = jnp.einsum("rb,bnd->rnd", rgcn_comp, rgcn_basis).reshape(
        N_REL * N_ENTITY, D)
    winv = 1.0 / jnp.maximum(cnt, 1.0)
    accE = _sc_ent_edges(weight, winv, gidx.astype(jnp.int32),
                         seg.astype(jnp.int32), dst.astype(jnp.int32))
    E = accE + rgcn_root + rgcn_bias

    dinv = 1.0 / jnp.sqrt(jnp.maximum(deg, 1.0))
    xw = word_embedding @ gcn_weight
    y = dinv[:, None] * xw
    accW = _sc_word_edges(y, ws.astype(jnp.int32), wd.astype(jnp.int32))
    wrep = dinv[:, None] * (accW + y) + gcn_bias

    def attn(table, Wa, b, idx):
        tvec = jnp.tanh(table @ Wa) @ b
        sc = tvec[idx]
        mask = idx != 0
        sc = jnp.where(mask, sc, -1e30)
        a = jax.nn.softmax(sc, axis=-1)
        return jnp.einsum("bl,bld->bd", a, table[idx])

    kg = attn(E, ent_attn_Wa, ent_attn_b, context_entities)
    wa = attn(wrep, word_attn_Wa, word_attn_b, context_words)
    g = jax.nn.sigmoid(jnp.concatenate([kg, wa], -1) @ gate_W + gate_b)
    u = g * kg + (1 - g) * wa
    return u @ E.T + rec_bias


# speedup vs baseline: 5.3982x; 2.2208x over previous
import dataclasses
import functools

import jax
import jax.numpy as jnp
from jax import lax
from jax.experimental import pallas as pl
from jax.experimental.pallas import tpu as pltpu
from jax.experimental.pallas import tpu_sc as plsc

N_ENTITY = 10000
N_WORDS = 20000
N_REL = 12
N_BASES = 8
D = 128
B = 1024
LE = 50
LW = 200
E_ENT = 320000
E_WORD = 320000

NC = 2
NS = 16
NW = NC * NS

_MESH = plsc.VectorSubcoreMesh(core_axis_name="c", subcore_axis_name="s")

_SC_PARAMS = pltpu.CompilerParams()
if "needs_layout_passes" in pltpu.CompilerParams.__dataclass_fields__:
    _SC_PARAMS = dataclasses.replace(_SC_PARAMS, needs_layout_passes=False)

CNT_Q = 7552
CNT_PAD = NS * CNT_Q
DEG_Q = 1280
DEG_PAD = NS * DEG_Q

CHUNK = 128


def _fill16(ref, val, n):
    @pl.loop(0, n, step=16)
    def _(i):
        ref[pl.ds(i, 16)] = jnp.full((16,), val, ref.dtype)


def _hist_kernel(seg_hbm, dstw_hbm, cnt_out, deg_out,
                 cnt_sp, deg_sp, zbuf, ones_v, idx_v):
    c = lax.axis_index("c")
    s = lax.axis_index("s")
    wid = c * NS + s

    _fill16(zbuf, 0.0, CNT_Q)
    pltpu.sync_copy(zbuf.at[pl.ds(0, CNT_Q)], cnt_sp.at[pl.ds(s * CNT_Q, CNT_Q)])
    pltpu.sync_copy(zbuf.at[pl.ds(0, DEG_Q)], deg_sp.at[pl.ds(s * DEG_Q, DEG_Q)])
    _fill16(ones_v, 1.0, CHUNK)
    plsc.subcore_barrier()

    n_chunks_ent = E_ENT // CHUNK
    n_chunks_word = E_WORD // CHUNK

    @pl.loop(wid, n_chunks_ent, step=NW)
    def _(j):
        pltpu.sync_copy(seg_hbm.at[pl.ds(j * CHUNK, CHUNK)], idx_v.at[0])
        pltpu.sync_copy(ones_v, cnt_sp.at[idx_v.at[0]], add=True)

    @pl.loop(wid, n_chunks_word, step=NW)
    def _(j):
        pltpu.sync_copy(dstw_hbm.at[pl.ds(j * CHUNK, CHUNK)], idx_v.at[0])
        pltpu.sync_copy(ones_v, deg_sp.at[idx_v.at[0]], add=True)

    plsc.subcore_barrier()

    pltpu.sync_copy(cnt_sp.at[pl.ds(s * CNT_Q, CNT_Q)],
                    cnt_out.at[c, pl.ds(s * CNT_Q, CNT_Q)])
    pltpu.sync_copy(deg_sp.at[pl.ds(s * DEG_Q, DEG_Q)],
                    deg_out.at[c, pl.ds(s * DEG_Q, DEG_Q)])


def _sc_histograms(seg_ent, dst_word):
    k = pl.kernel(
        _hist_kernel,
        out_type=(jax.ShapeDtypeStruct((NC, CNT_PAD), jnp.float32),
                  jax.ShapeDtypeStruct((NC, DEG_PAD), jnp.float32)),
        mesh=_MESH,
        scratch_types=[
            pltpu.VMEM_SHARED((CNT_PAD,), jnp.float32),
            pltpu.VMEM_SHARED((DEG_PAD,), jnp.float32),
            pltpu.VMEM((CNT_Q,), jnp.float32),
            pltpu.VMEM((CHUNK,), jnp.float32),
            pltpu.VMEM((1, CHUNK), jnp.int32),
        ],
    )
    cnt2, deg2 = k(seg_ent, dst_word)
    cnt = (cnt2[0] + cnt2[1])[:N_ENTITY * N_REL]
    deg = (deg2[0] + deg2[1])[:N_WORDS]
    return cnt, deg


ACC_E_Q = 640
ACC_E_PAD = NS * ACC_E_Q
ACC_W_Q = 640
ACC_W_PAD = NS * ACC_W_Q


def _zero_shared_rows(acc_sp, rows, s, quota):
    @pl.loop(0, CHUNK)
    def _(r):
        @pl.loop(0, D, step=16)
        def _(k):
            rows[r, pl.ds(k, 16)] = jnp.zeros((16,), jnp.float32)

    @pl.loop(0, quota, step=CHUNK)
    def _(r):
        pltpu.sync_copy(rows, acc_sp.at[pl.ds(s * quota + r, CHUNK)])


def _ent_edge_kernel(wflat_hbm, winv_hbm, gidx_hbm, seg_hbm, dst_hbm, acc_out,
                     acc_sp, rows, gidx_v, seg_v, dst_v, w_v, sem_r, sem_w):
    c = lax.axis_index("c")
    s = lax.axis_index("s")
    wid = c * NS + s

    _zero_shared_rows(acc_sp, rows, s, ACC_E_Q)
    plsc.subcore_barrier()

    n_chunks = E_ENT // CHUNK

    @pl.loop(wid, n_chunks, step=NW)
    def _(j):
        base = j * CHUNK
        pltpu.sync_copy(gidx_hbm.at[pl.ds(base, CHUNK)], gidx_v.at[0])
        pltpu.sync_copy(seg_hbm.at[pl.ds(base, CHUNK)], seg_v.at[0])
        pltpu.sync_copy(dst_hbm.at[pl.ds(base, CHUNK)], dst_v.at[0])
        cp_r = pltpu.async_copy(wflat_hbm.at[gidx_v.at[0]], rows, sem_r)
        cp_w = pltpu.async_copy(winv_hbm.at[seg_v.at[0]], w_v, sem_w)
        cp_w.wait()
        cp_r.wait()

        @pl.loop(0, CHUNK)
        def _(e):
            wsplat = plsc.load_gather(w_v, [jnp.full((16,), e, jnp.int32)])
            for k in range(D // 16):
                sl = rows.at[e, pl.ds(k * 16, 16)]
                sl[...] = sl[...] * wsplat

        pltpu.sync_copy(rows, acc_sp.at[dst_v.at[0]], add=True)

    plsc.subcore_barrier()

    @pl.loop(0, ACC_E_Q, step=CHUNK)
    def _(r):
        pltpu.sync_copy(acc_sp.at[pl.ds(s * ACC_E_Q + r, CHUNK)],
                        acc_out.at[c, pl.ds(s * ACC_E_Q + r, CHUNK)])


def _word_edge_kernel(y_hbm, src_hbm, dst_hbm, acc_out,
                      acc_sp, rows, src_v, dl_v, lidx_v, sem_r):
    c = lax.axis_index("c")
    s = lax.axis_index("s")

    _zero_shared_rows(acc_sp, rows, s, ACC_W_Q)
    plsc.subcore_barrier()

    n_chunks = E_WORD // CHUNK
    lo = c * N_ENTITY

    @pl.loop(s, n_chunks, step=NS)
    def _(j):
        base = j * CHUNK
        pltpu.sync_copy(src_hbm.at[pl.ds(base, CHUNK)], src_v.at[0])
        pltpu.sync_copy(dst_hbm.at[pl.ds(base, CHUNK)], dl_v)
        cp_r = pltpu.async_copy(y_hbm.at[src_v.at[0]], rows, sem_r)
        for k in range(CHUNK // 16):
            v = dl_v[pl.ds(k * 16, 16)] - lo
            inb = (v >= 0) & (v < 10000)
            trash = jnp.full((16,), 10000 + k * 16, jnp.int32) + lax.iota(
                jnp.int32, 16)
            lidx_v[pl.ds(k * 16, 16)] = jnp.where(inb, v, trash)
        cp_r.wait()
        pltpu.sync_copy(rows, acc_sp.at[lidx_v], add=True)

    plsc.subcore_barrier()

    @pl.loop(0, ACC_W_Q, step=CHUNK)
    def _(r):
        pltpu.sync_copy(acc_sp.at[pl.ds(s * ACC_W_Q + r, CHUNK)],
                        acc_out.at[c, pl.ds(s * ACC_W_Q + r, CHUNK)])


def _sc_ent_edges(wflat, winv, gidx, seg, dst):
    k = pl.kernel(
        _ent_edge_kernel,
        out_type=jax.ShapeDtypeStruct((NC, ACC_E_PAD, D), jnp.float32),
        mesh=_MESH,
        scratch_types=[
            pltpu.VMEM_SHARED((ACC_E_PAD, D), jnp.float32),
            pltpu.VMEM((CHUNK, D), jnp.float32),
            pltpu.VMEM((1, CHUNK), jnp.int32),
            pltpu.VMEM((1, CHUNK), jnp.int32),
            pltpu.VMEM((1, CHUNK), jnp.int32),
            pltpu.VMEM((CHUNK,), jnp.float32),
            pltpu.SemaphoreType.DMA,
            pltpu.SemaphoreType.DMA,
        ],
        compiler_params=_SC_PARAMS,
    )
    acc2 = k(wflat, winv, gidx, seg, dst)
    return (acc2[0] + acc2[1])[:N_ENTITY]


def _sc_word_edges(y, src_w, dst_w):
    k = pl.kernel(
        _word_edge_kernel,
        out_type=jax.ShapeDtypeStruct((NC, ACC_W_PAD, D), jnp.float32),
        mesh=_MESH,
        scratch_types=[
            pltpu.VMEM_SHARED((ACC_W_PAD, D), jnp.float32),
            pltpu.VMEM((CHUNK, D), jnp.float32),
            pltpu.VMEM((1, CHUNK), jnp.int32),
            pltpu.VMEM((CHUNK,), jnp.int32),
            pltpu.VMEM((CHUNK,), jnp.int32),
            pltpu.SemaphoreType.DMA,
        ],
        compiler_params=_SC_PARAMS,
    )
    acc2 = k(y, src_w, dst_w)
    return jnp.concatenate([acc2[0, :N_ENTITY], acc2[1, :N_ENTITY]], axis=0)


TVE_PAD = 10240
TVW_PAD = 20480
SE_N = B * LE
SW_N = B * LW


def _scores_kernel(tve_hbm, tvw_hbm, ce_hbm, cw_hbm, se_out, sw_out,
                   tve_sp, tvw_sp, idx_v, s_v, sem):
    c = lax.axis_index("c")
    s = lax.axis_index("s")
    wid = c * NS + s

    pltpu.sync_copy(tve_hbm.at[pl.ds(s * 640, 640)],
                    tve_sp.at[pl.ds(s * 640, 640)])
    pltpu.sync_copy(tvw_hbm.at[pl.ds(s * 1280, 1280)],
                    tvw_sp.at[pl.ds(s * 1280, 1280)])
    plsc.subcore_barrier()

    @pl.loop(wid, SE_N // CHUNK, step=NW)
    def _(j):
        base = j * CHUNK
        pltpu.sync_copy(ce_hbm.at[pl.ds(base, CHUNK)], idx_v.at[0])
        pltpu.async_copy(tve_sp.at[idx_v.at[0]], s_v, sem).wait()
        pltpu.sync_copy(s_v, se_out.at[pl.ds(base, CHUNK)])

    @pl.loop(wid, SW_N // CHUNK, step=NW)
    def _(j):
        base = j * CHUNK
        pltpu.sync_copy(cw_hbm.at[pl.ds(base, CHUNK)], idx_v.at[0])
        pltpu.async_copy(tvw_sp.at[idx_v.at[0]], s_v, sem).wait()
        pltpu.sync_copy(s_v, sw_out.at[pl.ds(base, CHUNK)])


def _sc_scores(tvec_e, tvec_w, ce_flat, cw_flat):
    k = pl.kernel(
        _scores_kernel,
        out_type=(jax.ShapeDtypeStruct((SE_N,), jnp.float32),
                  jax.ShapeDtypeStruct((SW_N,), jnp.float32)),
        mesh=_MESH,
        scratch_types=[
            pltpu.VMEM_SHARED((TVE_PAD,), jnp.float32),
            pltpu.VMEM_SHARED((TVW_PAD,), jnp.float32),
            pltpu.VMEM((1, CHUNK), jnp.int32),
            pltpu.VMEM((CHUNK,), jnp.float32),
            pltpu.SemaphoreType.DMA,
        ],
        compiler_params=_SC_PARAMS,
    )
    tve = jnp.pad(tvec_e, (0, TVE_PAD - N_ENTITY))
    tvw = jnp.pad(tvec_w, (0, TVW_PAD - N_WORDS))
    return k(tve, tvw, ce_flat, cw_flat)


def _attn_kernel(e_hbm, w_hbm, ce_hbm, ae_hbm, be_hbm, cw_hbm, aw_hbm,
                 bw_hbm, acce_out, accw_out,
                 acce_sp, accw_sp, rows, idx_v, b_v, a_v, sem_r, sem_w):
    c = lax.axis_index("c")
    s = lax.axis_index("s")
    wid = c * NS + s

    @pl.loop(0, CHUNK)
    def _(r):
        @pl.loop(0, D, step=16)
        def _(k):
            rows[r, pl.ds(k, 16)] = jnp.zeros((16,), jnp.float32)

    pltpu.sync_copy(rows.at[pl.ds(0, 64)], acce_sp.at[pl.ds(s * 64, 64)])
    pltpu.sync_copy(rows.at[pl.ds(0, 64)], accw_sp.at[pl.ds(s * 64, 64)])
    plsc.subcore_barrier()

    def pass_over(table_hbm, idx_hbm, a_hbm, bidx_hbm, acc_sp, n_pairs):
        @pl.loop(wid, n_pairs // CHUNK, step=NW)
        def _(j):
            base = j * CHUNK
            pltpu.sync_copy(idx_hbm.at[pl.ds(base, CHUNK)], idx_v.at[0])
            pltpu.sync_copy(bidx_hbm.at[pl.ds(base, CHUNK)], b_v.at[0])
            cp_a = pltpu.async_copy(a_hbm.at[pl.ds(base, CHUNK)], a_v, sem_w)
            cp_r = pltpu.async_copy(table_hbm.at[idx_v.at[0]], rows, sem_r)
            cp_a.wait()
            cp_r.wait()

            @pl.loop(0, CHUNK)
            def _(e):
                asplat = plsc.load_gather(a_v, [jnp.full((16,), e, jnp.int32)])
                for k in range(D // 16):
                    sl = rows.at[e, pl.ds(k * 16, 16)]
                    sl[...] = sl[...] * asplat

            pltpu.sync_copy(rows, acc_sp.at[b_v.at[0]], add=True)

    pass_over(e_hbm, ce_hbm, ae_hbm, be_hbm, acce_sp, SE_N)
    pass_over(w_hbm, cw_hbm, aw_hbm, bw_hbm, accw_sp, SW_N)
    plsc.subcore_barrier()

    pltpu.sync_copy(acce_sp.at[pl.ds(s * 64, 64)],
                    acce_out.at[c, pl.ds(s * 64, 64)])
    pltpu.sync_copy(accw_sp.at[pl.ds(s * 64, 64)],
                    accw_out.at[c, pl.ds(s * 64, 64)])


def _sc_attn(E, wrep, ce_flat, aE_flat, bE_flat, cw_flat, aW_flat, bW_flat):
    k = pl.kernel(
        _attn_kernel,
        out_type=(jax.ShapeDtypeStruct((NC, B, D), jnp.float32),
                  jax.ShapeDtypeStruct((NC, B, D), jnp.float32)),
        mesh=_MESH,
        scratch_types=[
            pltpu.VMEM_SHARED((B, D), jnp.float32),
            pltpu.VMEM_SHARED((B, D), jnp.float32),
            pltpu.VMEM((CHUNK, D), jnp.float32),
            pltpu.VMEM((1, CHUNK), jnp.int32),
            pltpu.VMEM((1, CHUNK), jnp.int32),
            pltpu.VMEM((CHUNK,), jnp.float32),
            pltpu.SemaphoreType.DMA,
            pltpu.SemaphoreType.DMA,
        ],
        compiler_params=_SC_PARAMS,
    )
    accE2, accW2 = k(E, wrep, ce_flat, aE_flat, bE_flat,
                     cw_flat, aW_flat, bW_flat)
    return accE2[0] + accE2[1], accW2[0] + accW2[1]


def kernel(context_entities, context_words, context_tokens,
           llm_compressed_tokens, edge_index, edge_type, word_edge_index,
           rgcn_basis, rgcn_comp, rgcn_root, rgcn_bias,
           word_embedding, gcn_weight, gcn_bias,
           ent_attn_Wa, ent_attn_b, word_attn_Wa, word_attn_b,
           gate_W, gate_b, rec_bias):
    src, dst = edge_index[0], edge_index[1]
    rel = edge_type
    seg = dst * N_REL + rel
    gidx = rel * N_ENTITY + src
    ws, wd = word_edge_index[0], word_edge_index[1]

    cnt, deg = _sc_histograms(seg.astype(jnp.int32), wd.astype(jnp.int32))
    deg = deg + 1.0

    weight = jnp.einsum("rb,bnd->rnd", rgcn_comp, rgcn_basis).reshape(
        N_REL * N_ENTITY, D)
    winv = 1.0 / jnp.maximum(cnt, 1.0)
    accE = _sc_ent_edges(weight, winv, gidx.astype(jnp.int32),
                         seg.astype(jnp.int32), dst.astype(jnp.int32))
    E = accE + rgcn_root + rgcn_bias

    dinv = 1.0 / jnp.sqrt(jnp.maximum(deg, 1.0))
    xw = word_embedding @ gcn_weight
    y = dinv[:, None] * xw
    accW = _sc_word_edges(y, ws.astype(jnp.int32), wd.astype(jnp.int32))
    wrep = dinv[:, None] * (accW + y) + gcn_bias

    tvec_e = jnp.tanh(E @ ent_attn_Wa) @ ent_attn_b
    tvec_w = jnp.tanh(wrep @ word_attn_Wa) @ word_attn_b
    ce_flat = context_entities.reshape(-1).astype(jnp.int32)
    cw_flat = context_words.reshape(-1).astype(jnp.int32)
    sE, sW = _sc_scores(tvec_e, tvec_w, ce_flat, cw_flat)

    def softmax_masked(s, idx, L):
        s = s.reshape(B, L)
        s = jnp.where(idx != 0, s, -1e30)
        return jax.nn.softmax(s, axis=-1)

    aE = softmax_masked(sE, context_entities, LE).reshape(-1)
    aW = softmax_masked(sW, context_words, LW).reshape(-1)
    bE_flat = jnp.repeat(jnp.arange(B, dtype=jnp.int32), LE)
    bW_flat = jnp.repeat(jnp.arange(B, dtype=jnp.int32), LW)
    kg, wa = _sc_attn(E, wrep, ce_flat, aE, bE_flat, cw_flat, aW, bW_flat)
    g = jax.nn.sigmoid(jnp.concatenate([kg, wa], -1) @ gate_W + gate_b)
    u = g * kg + (1 - g) * wa
    return u @ E.T + rec_bias


# speedup vs baseline: 7.0506x; 1.3061x over previous
import dataclasses
import functools

import jax
import jax.numpy as jnp
from jax import lax
from jax.experimental import pallas as pl
from jax.experimental.pallas import tpu as pltpu
from jax.experimental.pallas import tpu_sc as plsc

N_ENTITY = 10000
N_WORDS = 20000
N_REL = 12
N_BASES = 8
D = 128
B = 1024
LE = 50
LW = 200
E_ENT = 320000
E_WORD = 320000

NC = 2
NS = 16
NW = NC * NS

_MESH = plsc.VectorSubcoreMesh(core_axis_name="c", subcore_axis_name="s")

_SC_PARAMS = pltpu.CompilerParams()
if "needs_layout_passes" in pltpu.CompilerParams.__dataclass_fields__:
    _SC_PARAMS = dataclasses.replace(_SC_PARAMS, needs_layout_passes=False)

CNT_Q = 7552
CNT_PAD = NS * CNT_Q
DEG_Q = 1280
DEG_PAD = NS * DEG_Q

CHUNK = 128


def _fill16(ref, val, n):
    @pl.loop(0, n, step=16)
    def _(i):
        ref[pl.ds(i, 16)] = jnp.full((16,), val, ref.dtype)


def _hist_kernel(seg_hbm, dstw_hbm, cnt_out, deg_out,
                 cnt_sp, deg_sp, zbuf, ones_v, idx_v):
    c = lax.axis_index("c")
    s = lax.axis_index("s")
    wid = c * NS + s

    _fill16(zbuf, 0.0, CNT_Q)
    pltpu.sync_copy(zbuf.at[pl.ds(0, CNT_Q)], cnt_sp.at[pl.ds(s * CNT_Q, CNT_Q)])
    pltpu.sync_copy(zbuf.at[pl.ds(0, DEG_Q)], deg_sp.at[pl.ds(s * DEG_Q, DEG_Q)])
    _fill16(ones_v, 1.0, CHUNK)
    plsc.subcore_barrier()

    n_chunks_ent = E_ENT // CHUNK
    n_chunks_word = E_WORD // CHUNK

    @pl.loop(wid, n_chunks_ent, step=NW)
    def _(j):
        pltpu.sync_copy(seg_hbm.at[pl.ds(j * CHUNK, CHUNK)], idx_v.at[0])
        pltpu.sync_copy(ones_v, cnt_sp.at[idx_v.at[0]], add=True)

    @pl.loop(wid, n_chunks_word, step=NW)
    def _(j):
        pltpu.sync_copy(dstw_hbm.at[pl.ds(j * CHUNK, CHUNK)], idx_v.at[0])
        pltpu.sync_copy(ones_v, deg_sp.at[idx_v.at[0]], add=True)

    plsc.subcore_barrier()

    pltpu.sync_copy(cnt_sp.at[pl.ds(s * CNT_Q, CNT_Q)],
                    cnt_out.at[c, pl.ds(s * CNT_Q, CNT_Q)])
    pltpu.sync_copy(deg_sp.at[pl.ds(s * DEG_Q, DEG_Q)],
                    deg_out.at[c, pl.ds(s * DEG_Q, DEG_Q)])


def _sc_histograms(seg_ent, dst_word):
    k = pl.kernel(
        _hist_kernel,
        out_type=(jax.ShapeDtypeStruct((NC, CNT_PAD), jnp.float32),
                  jax.ShapeDtypeStruct((NC, DEG_PAD), jnp.float32)),
        mesh=_MESH,
        scratch_types=[
            pltpu.VMEM_SHARED((CNT_PAD,), jnp.float32),
            pltpu.VMEM_SHARED((DEG_PAD,), jnp.float32),
            pltpu.VMEM((CNT_Q,), jnp.float32),
            pltpu.VMEM((CHUNK,), jnp.float32),
            pltpu.VMEM((1, CHUNK), jnp.int32),
        ],
    )
    cnt2, deg2 = k(seg_ent, dst_word)
    cnt = (cnt2[0] + cnt2[1])[:N_ENTITY * N_REL]
    deg = (deg2[0] + deg2[1])[:N_WORDS]
    return cnt, deg


ACC_E_Q = 640
ACC_E_PAD = NS * ACC_E_Q
ACC_W_Q = 640
ACC_W_PAD = NS * ACC_W_Q


def _zero_shared_rows(acc_sp, rows, s, quota):
    @pl.loop(0, CHUNK)
    def _(r):
        @pl.loop(0, D, step=16)
        def _(k):
            rows[r, pl.ds(k, 16)] = jnp.zeros((16,), jnp.float32)

    @pl.loop(0, quota, step=CHUNK)
    def _(r):
        pltpu.sync_copy(rows, acc_sp.at[pl.ds(s * quota + r, CHUNK)])


def _scale_rows(rows, w_v):
    @pl.loop(0, CHUNK)
    def _(e):
        wsplat = plsc.load_gather(w_v, [jnp.full((16,), e, jnp.int32)])
        for k in range(D // 16):
            sl = rows.at[e, pl.ds(k * 16, 16)]
            sl[...] = sl[...] * wsplat


def _ent_edge_kernel(wflat_hbm, winv_hbm, gidx_hbm, seg_hbm, dst_hbm, acc_out,
                     acc_sp, rowsA, rowsB, gidxA, gidxB, segA, segB,
                     dstA, dstB, wA, wB, semA, semB, semWA, semWB, semS):
    c = lax.axis_index("c")
    s = lax.axis_index("s")
    wid = c * NS + s

    _zero_shared_rows(acc_sp, rowsA, s, ACC_E_Q)
    plsc.subcore_barrier()

    n_chunks = E_ENT // CHUNK
    main = n_chunks - n_chunks % (2 * NW)

    def load_and_gather(j, gidx_v, seg_v, dst_v, rows, w_v, sem_r, sem_w):
        base = j * CHUNK
        pltpu.sync_copy(gidx_hbm.at[pl.ds(base, CHUNK)], gidx_v.at[0])
        pltpu.sync_copy(seg_hbm.at[pl.ds(base, CHUNK)], seg_v.at[0])
        pltpu.sync_copy(dst_hbm.at[pl.ds(base, CHUNK)], dst_v.at[0])
        cp_r = pltpu.async_copy(wflat_hbm.at[gidx_v.at[0]], rows, sem_r)
        cp_w = pltpu.async_copy(winv_hbm.at[seg_v.at[0]], w_v, sem_w)
        return cp_r, cp_w

    @pl.loop(wid, main, step=2 * NW)
    def _(j):
        gA_r, gA_w = load_and_gather(j, gidxA, segA, dstA, rowsA, wA,
                                     semA, semWA)
        gB_r, gB_w = load_and_gather(j + NW, gidxB, segB, dstB, rowsB, wB,
                                     semB, semWB)
        gA_w.wait()
        gA_r.wait()
        _scale_rows(rowsA, wA)
        sA = pltpu.async_copy(rowsA, acc_sp.at[dstA.at[0]], semS, add=True)
        gB_w.wait()
        gB_r.wait()
        _scale_rows(rowsB, wB)
        sB = pltpu.async_copy(rowsB, acc_sp.at[dstB.at[0]], semS, add=True)
        sA.wait()
        sB.wait()

    @pl.when(wid < n_chunks - main)
    def _():
        gA_r, gA_w = load_and_gather(main + wid, gidxA, segA, dstA, rowsA, wA,
                                     semA, semWA)
        gA_w.wait()
        gA_r.wait()
        _scale_rows(rowsA, wA)
        pltpu.sync_copy(rowsA, acc_sp.at[dstA.at[0]], add=True)

    plsc.subcore_barrier()

    @pl.loop(0, ACC_E_Q, step=CHUNK)
    def _(r):
        pltpu.sync_copy(acc_sp.at[pl.ds(s * ACC_E_Q + r, CHUNK)],
                        acc_out.at[c, pl.ds(s * ACC_E_Q + r, CHUNK)])


def _word_edge_kernel(y_hbm, src_hbm, dst_hbm, acc_out,
                      acc_sp, rowsA, rowsB, srcA, srcB, dlA, dlB,
                      lidxA, lidxB, semA, semB, semS):
    c = lax.axis_index("c")
    s = lax.axis_index("s")

    _zero_shared_rows(acc_sp, rowsA, s, ACC_W_Q)
    plsc.subcore_barrier()

    n_chunks = E_WORD // CHUNK
    main = n_chunks - n_chunks % (2 * NS)
    lo = c * N_ENTITY

    def fixup(dl_v, lidx_v):
        for k in range(CHUNK // 16):
            v = dl_v[pl.ds(k * 16, 16)] - lo
            inb = (v >= 0) & (v < 10000)
            trash = jnp.full((16,), 10000 + k * 16, jnp.int32) + lax.iota(
                jnp.int32, 16)
            lidx_v[pl.ds(k * 16, 16)] = jnp.where(inb, v, trash)

    def load_and_gather(j, src_v, dl_v, rows, sem_r):
        base = j * CHUNK
        pltpu.sync_copy(src_hbm.at[pl.ds(base, CHUNK)], src_v.at[0])
        pltpu.sync_copy(dst_hbm.at[pl.ds(base, CHUNK)], dl_v)
        return pltpu.async_copy(y_hbm.at[src_v.at[0]], rows, sem_r)

    @pl.loop(s, main, step=2 * NS)
    def _(j):
        gA = load_and_gather(j, srcA, dlA, rowsA, semA)
        gB = load_and_gather(j + NS, srcB, dlB, rowsB, semB)
        fixup(dlA, lidxA)
        fixup(dlB, lidxB)
        gA.wait()
        sA = pltpu.async_copy(rowsA, acc_sp.at[lidxA], semS, add=True)
        gB.wait()
        sB = pltpu.async_copy(rowsB, acc_sp.at[lidxB], semS, add=True)
        sA.wait()
        sB.wait()

    @pl.when(s < n_chunks - main)
    def _():
        gA = load_and_gather(main + s, srcA, dlA, rowsA, semA)
        fixup(dlA, lidxA)
        gA.wait()
        pltpu.sync_copy(rowsA, acc_sp.at[lidxA], add=True)

    plsc.subcore_barrier()

    @pl.loop(0, ACC_W_Q, step=CHUNK)
    def _(r):
        pltpu.sync_copy(acc_sp.at[pl.ds(s * ACC_W_Q + r, CHUNK)],
                        acc_out.at[c, pl.ds(s * ACC_W_Q + r, CHUNK)])


def _sc_ent_edges(wflat, winv, gidx, seg, dst):
    k = pl.kernel(
        _ent_edge_kernel,
        out_type=jax.ShapeDtypeStruct((NC, ACC_E_PAD, D), jnp.float32),
        mesh=_MESH,
        scratch_types=[
            pltpu.VMEM_SHARED((ACC_E_PAD, D), jnp.float32),
            pltpu.VMEM((CHUNK, D), jnp.float32),
            pltpu.VMEM((CHUNK, D), jnp.float32),
            pltpu.VMEM((1, CHUNK), jnp.int32),
            pltpu.VMEM((1, CHUNK), jnp.int32),
            pltpu.VMEM((1, CHUNK), jnp.int32),
            pltpu.VMEM((1, CHUNK), jnp.int32),
            pltpu.VMEM((1, CHUNK), jnp.int32),
            pltpu.VMEM((1, CHUNK), jnp.int32),
            pltpu.VMEM((CHUNK,), jnp.float32),
            pltpu.VMEM((CHUNK,), jnp.float32),
            pltpu.SemaphoreType.DMA,
            pltpu.SemaphoreType.DMA,
            pltpu.SemaphoreType.DMA,
            pltpu.SemaphoreType.DMA,
            pltpu.SemaphoreType.DMA,
        ],
        compiler_params=_SC_PARAMS,
    )
    acc2 = k(wflat, winv, gidx, seg, dst)
    return (acc2[0] + acc2[1])[:N_ENTITY]


def _sc_word_edges(y, src_w, dst_w):
    k = pl.kernel(
        _word_edge_kernel,
        out_type=jax.ShapeDtypeStruct((NC, ACC_W_PAD, D), jnp.float32),
        mesh=_MESH,
        scratch_types=[
            pltpu.VMEM_SHARED((ACC_W_PAD, D), jnp.float32),
            pltpu.VMEM((CHUNK, D), jnp.float32),
            pltpu.VMEM((CHUNK, D), jnp.float32),
            pltpu.VMEM((1, CHUNK), jnp.int32),
            pltpu.VMEM((1, CHUNK), jnp.int32),
            pltpu.VMEM((CHUNK,), jnp.int32),
            pltpu.VMEM((CHUNK,), jnp.int32),
            pltpu.VMEM((CHUNK,), jnp.int32),
            pltpu.VMEM((CHUNK,), jnp.int32),
            pltpu.SemaphoreType.DMA,
            pltpu.SemaphoreType.DMA,
            pltpu.SemaphoreType.DMA,
        ],
        compiler_params=_SC_PARAMS,
    )
    acc2 = k(y, src_w, dst_w)
    return jnp.concatenate([acc2[0, :N_ENTITY], acc2[1, :N_ENTITY]], axis=0)


TVE_PAD = 10240
TVW_PAD = 20480
SE_N = B * LE
SW_N = B * LW


def _scores_kernel(tve_hbm, tvw_hbm, ce_hbm, cw_hbm, se_out, sw_out,
                   tve_sp, tvw_sp, idx_v, s_v, sem):
    c = lax.axis_index("c")
    s = lax.axis_index("s")
    wid = c * NS + s

    pltpu.sync_copy(tve_hbm.at[pl.ds(s * 640, 640)],
                    tve_sp.at[pl.ds(s * 640, 640)])
    pltpu.sync_copy(tvw_hbm.at[pl.ds(s * 1280, 1280)],
                    tvw_sp.at[pl.ds(s * 1280, 1280)])
    plsc.subcore_barrier()

    @pl.loop(wid, SE_N // CHUNK, step=NW)
    def _(j):
        base = j * CHUNK
        pltpu.sync_copy(ce_hbm.at[pl.ds(base, CHUNK)], idx_v.at[0])
        pltpu.async_copy(tve_sp.at[idx_v.at[0]], s_v, sem).wait()
        pltpu.sync_copy(s_v, se_out.at[pl.ds(base, CHUNK)])

    @pl.loop(wid, SW_N // CHUNK, step=NW)
    def _(j):
        base = j * CHUNK
        pltpu.sync_copy(cw_hbm.at[pl.ds(base, CHUNK)], idx_v.at[0])
        pltpu.async_copy(tvw_sp.at[idx_v.at[0]], s_v, sem).wait()
        pltpu.sync_copy(s_v, sw_out.at[pl.ds(base, CHUNK)])


def _sc_scores(tvec_e, tvec_w, ce_flat, cw_flat):
    k = pl.kernel(
        _scores_kernel,
        out_type=(jax.ShapeDtypeStruct((SE_N,), jnp.float32),
                  jax.ShapeDtypeStruct((SW_N,), jnp.float32)),
        mesh=_MESH,
        scratch_types=[
            pltpu.VMEM_SHARED((TVE_PAD,), jnp.float32),
            pltpu.VMEM_SHARED((TVW_PAD,), jnp.float32),
            pltpu.VMEM((1, CHUNK), jnp.int32),
            pltpu.VMEM((CHUNK,), jnp.float32),
            pltpu.SemaphoreType.DMA,
        ],
        compiler_params=_SC_PARAMS,
    )
    tve = jnp.pad(tvec_e, (0, TVE_PAD - N_ENTITY))
    tvw = jnp.pad(tvec_w, (0, TVW_PAD - N_WORDS))
    return k(tve, tvw, ce_flat, cw_flat)


def _attn_kernel(e_hbm, w_hbm, ce_hbm, ae_hbm, be_hbm, cw_hbm, aw_hbm,
                 bw_hbm, acce_out, accw_out,
                 acce_sp, accw_sp, rowsA, rowsB, idxA, idxB, bA, bB,
                 aA, aB, semA, semB, semWA, semWB, semS):
    c = lax.axis_index("c")
    s = lax.axis_index("s")
    wid = c * NS + s

    @pl.loop(0, CHUNK)
    def _(r):
        @pl.loop(0, D, step=16)
        def _(k):
            rowsA[r, pl.ds(k, 16)] = jnp.zeros((16,), jnp.float32)

    pltpu.sync_copy(rowsA.at[pl.ds(0, 64)], acce_sp.at[pl.ds(s * 64, 64)])
    pltpu.sync_copy(rowsA.at[pl.ds(0, 64)], accw_sp.at[pl.ds(s * 64, 64)])
    plsc.subcore_barrier()

    def pass_over(table_hbm, idx_hbm, a_hbm, bidx_hbm, acc_sp, n_pairs):
        n_chunks = n_pairs // CHUNK
        main = n_chunks - n_chunks % (2 * NW)

        def load_and_gather(j, idx_v, b_v, rows, a_v, sem_r, sem_w):
            base = j * CHUNK
            pltpu.sync_copy(idx_hbm.at[pl.ds(base, CHUNK)], idx_v.at[0])
            pltpu.sync_copy(bidx_hbm.at[pl.ds(base, CHUNK)], b_v.at[0])
            cp_a = pltpu.async_copy(a_hbm.at[pl.ds(base, CHUNK)], a_v, sem_w)
            cp_r = pltpu.async_copy(table_hbm.at[idx_v.at[0]], rows, sem_r)
            return cp_r, cp_a

        @pl.loop(wid, main, step=2 * NW)
        def _(j):
            gA_r, gA_a = load_and_gather(j, idxA, bA, rowsA, aA, semA, semWA)
            gB_r, gB_a = load_and_gather(j + NW, idxB, bB, rowsB, aB,
                                         semB, semWB)
            gA_a.wait()
            gA_r.wait()
            _scale_rows(rowsA, aA)
            sA = pltpu.async_copy(rowsA, acc_sp.at[bA.at[0]], semS, add=True)
            gB_a.wait()
            gB_r.wait()
            _scale_rows(rowsB, aB)
            sB = pltpu.async_copy(rowsB, acc_sp.at[bB.at[0]], semS, add=True)
            sA.wait()
            sB.wait()

        @pl.when(wid < n_chunks - main)
        def _():
            gA_r, gA_a = load_and_gather(main + wid, idxA, bA, rowsA, aA,
                                         semA, semWA)
            gA_a.wait()
            gA_r.wait()
            _scale_rows(rowsA, aA)
            pltpu.sync_copy(rowsA, acc_sp.at[bA.at[0]], add=True)

    pass_over(e_hbm, ce_hbm, ae_hbm, be_hbm, acce_sp, SE_N)
    pass_over(w_hbm, cw_hbm, aw_hbm, bw_hbm, accw_sp, SW_N)
    plsc.subcore_barrier()

    pltpu.sync_copy(acce_sp.at[pl.ds(s * 64, 64)],
                    acce_out.at[c, pl.ds(s * 64, 64)])
    pltpu.sync_copy(accw_sp.at[pl.ds(s * 64, 64)],
                    accw_out.at[c, pl.ds(s * 64, 64)])


def _sc_attn(E, wrep, ce_flat, aE_flat, bE_flat, cw_flat, aW_flat, bW_flat):
    k = pl.kernel(
        _attn_kernel,
        out_type=(jax.ShapeDtypeStruct((NC, B, D), jnp.float32),
                  jax.ShapeDtypeStruct((NC, B, D), jnp.float32)),
        mesh=_MESH,
        scratch_types=[
            pltpu.VMEM_SHARED((B, D), jnp.float32),
            pltpu.VMEM_SHARED((B, D), jnp.float32),
            pltpu.VMEM((CHUNK, D), jnp.float32),
            pltpu.VMEM((CHUNK, D), jnp.float32),
            pltpu.VMEM((1, CHUNK), jnp.int32),
            pltpu.VMEM((1, CHUNK), jnp.int32),
            pltpu.VMEM((1, CHUNK), jnp.int32),
            pltpu.VMEM((1, CHUNK), jnp.int32),
            pltpu.VMEM((CHUNK,), jnp.float32),
            pltpu.VMEM((CHUNK,), jnp.float32),
            pltpu.SemaphoreType.DMA,
            pltpu.SemaphoreType.DMA,
            pltpu.SemaphoreType.DMA,
            pltpu.SemaphoreType.DMA,
            pltpu.SemaphoreType.DMA,
        ],
        compiler_params=_SC_PARAMS,
    )
    accE2, accW2 = k(E, wrep, ce_flat, aE_flat, bE_flat,
                     cw_flat, aW_flat, bW_flat)
    return accE2[0] + accE2[1], accW2[0] + accW2[1]


def kernel(context_entities, context_words, context_tokens,
           llm_compressed_tokens, edge_index, edge_type, word_edge_index,
           rgcn_basis, rgcn_comp, rgcn_root, rgcn_bias,
           word_embedding, gcn_weight, gcn_bias,
           ent_attn_Wa, ent_attn_b, word_attn_Wa, word_attn_b,
           gate_W, gate_b, rec_bias):
    src, dst = edge_index[0], edge_index[1]
    rel = edge_type
    seg = dst * N_REL + rel
    gidx = rel * N_ENTITY + src
    ws, wd = word_edge_index[0], word_edge_index[1]

    cnt, deg = _sc_histograms(seg.astype(jnp.int32), wd.astype(jnp.int32))
    deg = deg + 1.0

    weight = jnp.einsum("rb,bnd->rnd", rgcn_comp, rgcn_basis).reshape(
        N_REL * N_ENTITY, D)
    winv = 1.0 / jnp.maximum(cnt, 1.0)
    accE = _sc_ent_edges(weight, winv, gidx.astype(jnp.int32),
                         seg.astype(jnp.int32), dst.astype(jnp.int32))
    E = accE + rgcn_root + rgcn_bias

    dinv = 1.0 / jnp.sqrt(jnp.maximum(deg, 1.0))
    xw = word_embedding @ gcn_weight
    y = dinv[:, None] * xw
    accW = _sc_word_edges(y, ws.astype(jnp.int32), wd.astype(jnp.int32))
    wrep = dinv[:, None] * (accW + y) + gcn_bias

    tvec_e = jnp.tanh(E @ ent_attn_Wa) @ ent_attn_b
    tvec_w = jnp.tanh(wrep @ word_attn_Wa) @ word_attn_b
    ce_flat = context_entities.reshape(-1).astype(jnp.int32)
    cw_flat = context_words.reshape(-1).astype(jnp.int32)
    sE, sW = _sc_scores(tvec_e, tvec_w, ce_flat, cw_flat)

    def softmax_masked(s, idx, L):
        s = s.reshape(B, L)
        s = jnp.where(idx != 0, s, -1e30)
        return jax.nn.softmax(s, axis=-1)

    aE = softmax_masked(sE, context_entities, LE).reshape(-1)
    aW = softmax_masked(sW, context_words, LW).reshape(-1)
    bE_flat = jnp.repeat(jnp.arange(B, dtype=jnp.int32), LE)
    bW_flat = jnp.repeat(jnp.arange(B, dtype=jnp.int32), LW)
    kg, wa = _sc_attn(E, wrep, ce_flat, aE, bE_flat, cw_flat, aW, bW_flat)
    g = jax.nn.sigmoid(jnp.concatenate([kg, wa], -1) @ gate_W + gate_b)
    u = g * kg + (1 - g) * wa
    return u @ E.T + rec_bias


# speedup vs baseline: 7.5743x; 1.0743x over previous
import dataclasses
import functools

import jax
import jax.numpy as jnp
from jax import lax
from jax.experimental import pallas as pl
from jax.experimental.pallas import tpu as pltpu
from jax.experimental.pallas import tpu_sc as plsc

N_ENTITY = 10000
N_WORDS = 20000
N_REL = 12
N_BASES = 8
D = 128
B = 1024
LE = 50
LW = 200
E_ENT = 320000
E_WORD = 320000

NC = 2
NS = 16
NW = NC * NS

_MESH = plsc.VectorSubcoreMesh(core_axis_name="c", subcore_axis_name="s")

_SC_PARAMS = pltpu.CompilerParams()
if "needs_layout_passes" in pltpu.CompilerParams.__dataclass_fields__:
    _SC_PARAMS = dataclasses.replace(_SC_PARAMS, needs_layout_passes=False)

CNT_Q = 7552
CNT_PAD = NS * CNT_Q
DEG_Q = 1280
DEG_PAD = NS * DEG_Q

CHUNK = 128


def _fill16(ref, val, n):
    @pl.loop(0, n, step=16)
    def _(i):
        ref[pl.ds(i, 16)] = jnp.full((16,), val, ref.dtype)


def _hist_kernel(seg_hbm, dstw_hbm, cnt_out, deg_out,
                 cnt_sp, deg_sp, zbuf, ones_v, idxA, idxB,
                 semA, semB, semS):
    c = lax.axis_index("c")
    s = lax.axis_index("s")
    wid = c * NS + s

    _fill16(zbuf, 0.0, CNT_Q)
    pltpu.sync_copy(zbuf.at[pl.ds(0, CNT_Q)], cnt_sp.at[pl.ds(s * CNT_Q, CNT_Q)])
    pltpu.sync_copy(zbuf.at[pl.ds(0, DEG_Q)], deg_sp.at[pl.ds(s * DEG_Q, DEG_Q)])
    _fill16(ones_v, 1.0, CHUNK)
    plsc.subcore_barrier()

    def hist_pass(idx_hbm, acc_sp, n_chunks):
        main = n_chunks - n_chunks % (2 * NW)

        @pl.loop(wid, main, step=2 * NW)
        def _(j):
            gA = pltpu.async_copy(idx_hbm.at[pl.ds(j * CHUNK, CHUNK)],
                                  idxA.at[0], semA)
            gB = pltpu.async_copy(idx_hbm.at[pl.ds((j + NW) * CHUNK, CHUNK)],
                                  idxB.at[0], semB)
            gA.wait()
            sA = pltpu.async_copy(ones_v, acc_sp.at[idxA.at[0]], semS,
                                  add=True)
            gB.wait()
            sB = pltpu.async_copy(ones_v, acc_sp.at[idxB.at[0]], semS,
                                  add=True)
            sA.wait()
            sB.wait()

        @pl.when(wid < n_chunks - main)
        def _():
            base = (main + wid) * CHUNK
            pltpu.sync_copy(idx_hbm.at[pl.ds(base, CHUNK)], idxA.at[0])
            pltpu.sync_copy(ones_v, acc_sp.at[idxA.at[0]], add=True)

    hist_pass(seg_hbm, cnt_sp, E_ENT // CHUNK)
    hist_pass(dstw_hbm, deg_sp, E_WORD // CHUNK)

    plsc.subcore_barrier()

    pltpu.sync_copy(cnt_sp.at[pl.ds(s * CNT_Q, CNT_Q)],
                    cnt_out.at[c, pl.ds(s * CNT_Q, CNT_Q)])
    pltpu.sync_copy(deg_sp.at[pl.ds(s * DEG_Q, DEG_Q)],
                    deg_out.at[c, pl.ds(s * DEG_Q, DEG_Q)])


def _sc_histograms(seg_ent, dst_word):
    k = pl.kernel(
        _hist_kernel,
        out_type=(jax.ShapeDtypeStruct((NC, CNT_PAD), jnp.float32),
                  jax.ShapeDtypeStruct((NC, DEG_PAD), jnp.float32)),
        mesh=_MESH,
        scratch_types=[
            pltpu.VMEM_SHARED((CNT_PAD,), jnp.float32),
            pltpu.VMEM_SHARED((DEG_PAD,), jnp.float32),
            pltpu.VMEM((CNT_Q,), jnp.float32),
            pltpu.VMEM((CHUNK,), jnp.float32),
            pltpu.VMEM((1, CHUNK), jnp.int32),
            pltpu.VMEM((1, CHUNK), jnp.int32),
            pltpu.SemaphoreType.DMA,
            pltpu.SemaphoreType.DMA,
            pltpu.SemaphoreType.DMA,
        ],
        compiler_params=_SC_PARAMS,
    )
    cnt2, deg2 = k(seg_ent, dst_word)
    cnt = (cnt2[0] + cnt2[1])[:N_ENTITY * N_REL]
    deg = (deg2[0] + deg2[1])[:N_WORDS]
    return cnt, deg


ACC_E_Q = 640
ACC_E_PAD = NS * ACC_E_Q
ACC_W_Q = 640
ACC_W_PAD = NS * ACC_W_Q


def _zero_shared_rows(acc_sp, rows, s, quota):
    @pl.loop(0, CHUNK)
    def _(r):
        @pl.loop(0, D, step=16)
        def _(k):
            rows[r, pl.ds(k, 16)] = jnp.zeros((16,), jnp.float32)

    @pl.loop(0, quota, step=CHUNK)
    def _(r):
        pltpu.sync_copy(rows, acc_sp.at[pl.ds(s * quota + r, CHUNK)])


def _scale_rows(rows, w_v):
    @pl.loop(0, CHUNK)
    def _(e):
        wsplat = plsc.load_gather(w_v, [jnp.full((16,), e, jnp.int32)])
        for k in range(D // 16):
            sl = rows.at[e, pl.ds(k * 16, 16)]
            sl[...] = sl[...] * wsplat


def _ent_edge_kernel(wflat_hbm, winv_hbm, gidx_hbm, seg_hbm, dst_hbm, acc_out,
                     acc_sp, rowsA, rowsB, gidxA, gidxB, segA, segB,
                     dstA, dstB, wA, wB, semA, semB, semWA, semWB, semS):
    c = lax.axis_index("c")
    s = lax.axis_index("s")
    wid = c * NS + s

    _zero_shared_rows(acc_sp, rowsA, s, ACC_E_Q)
    plsc.subcore_barrier()

    n_chunks = E_ENT // CHUNK
    main = n_chunks - n_chunks % (2 * NW)

    def load_and_gather(j, gidx_v, seg_v, dst_v, rows, w_v, sem_r, sem_w):
        base = j * CHUNK
        pltpu.sync_copy(gidx_hbm.at[pl.ds(base, CHUNK)], gidx_v.at[0])
        pltpu.sync_copy(seg_hbm.at[pl.ds(base, CHUNK)], seg_v.at[0])
        pltpu.sync_copy(dst_hbm.at[pl.ds(base, CHUNK)], dst_v.at[0])
        cp_r = pltpu.async_copy(wflat_hbm.at[gidx_v.at[0]], rows, sem_r)
        cp_w = pltpu.async_copy(winv_hbm.at[seg_v.at[0]], w_v, sem_w)
        return cp_r, cp_w

    @pl.loop(wid, main, step=2 * NW)
    def _(j):
        gA_r, gA_w = load_and_gather(j, gidxA, segA, dstA, rowsA, wA,
                                     semA, semWA)
        gB_r, gB_w = load_and_gather(j + NW, gidxB, segB, dstB, rowsB, wB,
                                     semB, semWB)
        gA_w.wait()
        gA_r.wait()
        _scale_rows(rowsA, wA)
        sA = pltpu.async_copy(rowsA, acc_sp.at[dstA.at[0]], semS, add=True)
        gB_w.wait()
        gB_r.wait()
        _scale_rows(rowsB, wB)
        sB = pltpu.async_copy(rowsB, acc_sp.at[dstB.at[0]], semS, add=True)
        sA.wait()
        sB.wait()

    @pl.when(wid < n_chunks - main)
    def _():
        gA_r, gA_w = load_and_gather(main + wid, gidxA, segA, dstA, rowsA, wA,
                                     semA, semWA)
        gA_w.wait()
        gA_r.wait()
        _scale_rows(rowsA, wA)
        pltpu.sync_copy(rowsA, acc_sp.at[dstA.at[0]], add=True)

    plsc.subcore_barrier()

    @pl.loop(0, ACC_E_Q, step=CHUNK)
    def _(r):
        pltpu.sync_copy(acc_sp.at[pl.ds(s * ACC_E_Q + r, CHUNK)],
                        acc_out.at[c, pl.ds(s * ACC_E_Q + r, CHUNK)])


def _word_edge_kernel(y_hbm, src_hbm, dst_hbm, acc_out,
                      acc_sp, rowsA, rowsB, srcA, srcB, dlA, dlB,
                      lidxA, lidxB, semA, semB, semS):
    c = lax.axis_index("c")
    s = lax.axis_index("s")

    _zero_shared_rows(acc_sp, rowsA, s, ACC_W_Q)
    plsc.subcore_barrier()

    n_chunks = E_WORD // CHUNK
    main = n_chunks - n_chunks % (2 * NS)
    lo = c * N_ENTITY

    def fixup(dl_v, lidx_v):
        for k in range(CHUNK // 16):
            v = dl_v[pl.ds(k * 16, 16)] - lo
            inb = (v >= 0) & (v < 10000)
            trash = jnp.full((16,), 10000 + k * 16, jnp.int32) + lax.iota(
                jnp.int32, 16)
            lidx_v[pl.ds(k * 16, 16)] = jnp.where(inb, v, trash)

    def load_and_gather(j, src_v, dl_v, rows, sem_r):
        base = j * CHUNK
        pltpu.sync_copy(src_hbm.at[pl.ds(base, CHUNK)], src_v.at[0])
        pltpu.sync_copy(dst_hbm.at[pl.ds(base, CHUNK)], dl_v)
        return pltpu.async_copy(y_hbm.at[src_v.at[0]], rows, sem_r)

    @pl.loop(s, main, step=2 * NS)
    def _(j):
        gA = load_and_gather(j, srcA, dlA, rowsA, semA)
        gB = load_and_gather(j + NS, srcB, dlB, rowsB, semB)
        fixup(dlA, lidxA)
        fixup(dlB, lidxB)
        gA.wait()
        sA = pltpu.async_copy(rowsA, acc_sp.at[lidxA], semS, add=True)
        gB.wait()
        sB = pltpu.async_copy(rowsB, acc_sp.at[lidxB], semS, add=True)
        sA.wait()
        sB.wait()

    @pl.when(s < n_chunks - main)
    def _():
        gA = load_and_gather(main + s, srcA, dlA, rowsA, semA)
        fixup(dlA, lidxA)
        gA.wait()
        pltpu.sync_copy(rowsA, acc_sp.at[lidxA], add=True)

    plsc.subcore_barrier()

    @pl.loop(0, ACC_W_Q, step=CHUNK)
    def _(r):
        pltpu.sync_copy(acc_sp.at[pl.ds(s * ACC_W_Q + r, CHUNK)],
                        acc_out.at[c, pl.ds(s * ACC_W_Q + r, CHUNK)])


def _sc_ent_edges(wflat, winv, gidx, seg, dst):
    k = pl.kernel(
        _ent_edge_kernel,
        out_type=jax.ShapeDtypeStruct((NC, ACC_E_PAD, D), jnp.float32),
        mesh=_MESH,
        scratch_types=[
            pltpu.VMEM_SHARED((ACC_E_PAD, D), jnp.float32),
            pltpu.VMEM((CHUNK, D), jnp.float32),
            pltpu.VMEM((CHUNK, D), jnp.float32),
            pltpu.VMEM((1, CHUNK), jnp.int32),
            pltpu.VMEM((1, CHUNK), jnp.int32),
            pltpu.VMEM((1, CHUNK), jnp.int32),
            pltpu.VMEM((1, CHUNK), jnp.int32),
            pltpu.VMEM((1, CHUNK), jnp.int32),
            pltpu.VMEM((1, CHUNK), jnp.int32),
            pltpu.VMEM((CHUNK,), jnp.float32),
            pltpu.VMEM((CHUNK,), jnp.float32),
            pltpu.SemaphoreType.DMA,
            pltpu.SemaphoreType.DMA,
            pltpu.SemaphoreType.DMA,
            pltpu.SemaphoreType.DMA,
            pltpu.SemaphoreType.DMA,
        ],
        compiler_params=_SC_PARAMS,
    )
    acc2 = k(wflat, winv, gidx, seg, dst)
    return (acc2[0] + acc2[1])[:N_ENTITY]


def _sc_word_edges(y, src_w, dst_w):
    k = pl.kernel(
        _word_edge_kernel,
        out_type=jax.ShapeDtypeStruct((NC, ACC_W_PAD, D), jnp.float32),
        mesh=_MESH,
        scratch_types=[
            pltpu.VMEM_SHARED((ACC_W_PAD, D), jnp.float32),
            pltpu.VMEM((CHUNK, D), jnp.float32),
            pltpu.VMEM((CHUNK, D), jnp.float32),
            pltpu.VMEM((1, CHUNK), jnp.int32),
            pltpu.VMEM((1, CHUNK), jnp.int32),
            pltpu.VMEM((CHUNK,), jnp.int32),
            pltpu.VMEM((CHUNK,), jnp.int32),
            pltpu.VMEM((CHUNK,), jnp.int32),
            pltpu.VMEM((CHUNK,), jnp.int32),
            pltpu.SemaphoreType.DMA,
            pltpu.SemaphoreType.DMA,
            pltpu.SemaphoreType.DMA,
        ],
        compiler_params=_SC_PARAMS,
    )
    acc2 = k(y, src_w, dst_w)
    return jnp.concatenate([acc2[0, :N_ENTITY], acc2[1, :N_ENTITY]], axis=0)


TVE_PAD = 10240
TVW_PAD = 20480
SE_N = B * LE
SW_N = B * LW


def _scores_kernel(tve_hbm, tvw_hbm, ce_hbm, cw_hbm, se_out, sw_out,
                   tve_sp, tvw_sp, idx_v, s_v, sem):
    c = lax.axis_index("c")
    s = lax.axis_index("s")
    wid = c * NS + s

    pltpu.sync_copy(tve_hbm.at[pl.ds(s * 640, 640)],
                    tve_sp.at[pl.ds(s * 640, 640)])
    pltpu.sync_copy(tvw_hbm.at[pl.ds(s * 1280, 1280)],
                    tvw_sp.at[pl.ds(s * 1280, 1280)])
    plsc.subcore_barrier()

    @pl.loop(wid, SE_N // CHUNK, step=NW)
    def _(j):
        base = j * CHUNK
        pltpu.sync_copy(ce_hbm.at[pl.ds(base, CHUNK)], idx_v.at[0])
        pltpu.async_copy(tve_sp.at[idx_v.at[0]], s_v, sem).wait()
        pltpu.sync_copy(s_v, se_out.at[pl.ds(base, CHUNK)])

    @pl.loop(wid, SW_N // CHUNK, step=NW)
    def _(j):
        base = j * CHUNK
        pltpu.sync_copy(cw_hbm.at[pl.ds(base, CHUNK)], idx_v.at[0])
        pltpu.async_copy(tvw_sp.at[idx_v.at[0]], s_v, sem).wait()
        pltpu.sync_copy(s_v, sw_out.at[pl.ds(base, CHUNK)])


def _sc_scores(tvec_e, tvec_w, ce_flat, cw_flat):
    k = pl.kernel(
        _scores_kernel,
        out_type=(jax.ShapeDtypeStruct((SE_N,), jnp.float32),
                  jax.ShapeDtypeStruct((SW_N,), jnp.float32)),
        mesh=_MESH,
        scratch_types=[
            pltpu.VMEM_SHARED((TVE_PAD,), jnp.float32),
            pltpu.VMEM_SHARED((TVW_PAD,), jnp.float32),
            pltpu.VMEM((1, CHUNK), jnp.int32),
            pltpu.VMEM((CHUNK,), jnp.float32),
            pltpu.SemaphoreType.DMA,
        ],
        compiler_params=_SC_PARAMS,
    )
    tve = jnp.pad(tvec_e, (0, TVE_PAD - N_ENTITY))
    tvw = jnp.pad(tvec_w, (0, TVW_PAD - N_WORDS))
    return k(tve, tvw, ce_flat, cw_flat)


def _attn_kernel(e_hbm, w_hbm, ce_hbm, ae_hbm, be_hbm, cw_hbm, aw_hbm,
                 bw_hbm, acce_out, accw_out,
                 acce_sp, accw_sp, rowsA, rowsB, idxA, idxB, bA, bB,
                 aA, aB, semA, semB, semWA, semWB, semS):
    c = lax.axis_index("c")
    s = lax.axis_index("s")
    wid = c * NS + s

    @pl.loop(0, CHUNK)
    def _(r):
        @pl.loop(0, D, step=16)
        def _(k):
            rowsA[r, pl.ds(k, 16)] = jnp.zeros((16,), jnp.float32)

    pltpu.sync_copy(rowsA.at[pl.ds(0, 64)], acce_sp.at[pl.ds(s * 64, 64)])
    pltpu.sync_copy(rowsA.at[pl.ds(0, 64)], accw_sp.at[pl.ds(s * 64, 64)])
    plsc.subcore_barrier()

    def pass_over(table_hbm, idx_hbm, a_hbm, bidx_hbm, acc_sp, n_pairs):
        n_chunks = n_pairs // CHUNK
        main = n_chunks - n_chunks % (2 * NW)

        def load_and_gather(j, idx_v, b_v, rows, a_v, sem_r, sem_w):
            base = j * CHUNK
            pltpu.sync_copy(idx_hbm.at[pl.ds(base, CHUNK)], idx_v.at[0])
            pltpu.sync_copy(bidx_hbm.at[pl.ds(base, CHUNK)], b_v.at[0])
            cp_a = pltpu.async_copy(a_hbm.at[pl.ds(base, CHUNK)], a_v, sem_w)
            cp_r = pltpu.async_copy(table_hbm.at[idx_v.at[0]], rows, sem_r)
            return cp_r, cp_a

        @pl.loop(wid, main, step=2 * NW)
        def _(j):
            gA_r, gA_a = load_and_gather(j, idxA, bA, rowsA, aA, semA, semWA)
            gB_r, gB_a = load_and_gather(j + NW, idxB, bB, rowsB, aB,
                                         semB, semWB)
            gA_a.wait()
            gA_r.wait()
            _scale_rows(rowsA, aA)
            sA = pltpu.async_copy(rowsA, acc_sp.at[bA.at[0]], semS, add=True)
            gB_a.wait()
            gB_r.wait()
            _scale_rows(rowsB, aB)
            sB = pltpu.async_copy(rowsB, acc_sp.at[bB.at[0]], semS, add=True)
            sA.wait()
            sB.wait()

        @pl.when(wid < n_chunks - main)
        def _():
            gA_r, gA_a = load_and_gather(main + wid, idxA, bA, rowsA, aA,
                                         semA, semWA)
            gA_a.wait()
            gA_r.wait()
            _scale_rows(rowsA, aA)
            pltpu.sync_copy(rowsA, acc_sp.at[bA.at[0]], add=True)

    pass_over(e_hbm, ce_hbm, ae_hbm, be_hbm, acce_sp, SE_N)
    pass_over(w_hbm, cw_hbm, aw_hbm, bw_hbm, accw_sp, SW_N)
    plsc.subcore_barrier()

    pltpu.sync_copy(acce_sp.at[pl.ds(s * 64, 64)],
                    acce_out.at[c, pl.ds(s * 64, 64)])
    pltpu.sync_copy(accw_sp.at[pl.ds(s * 64, 64)],
                    accw_out.at[c, pl.ds(s * 64, 64)])


def _sc_attn(E, wrep, ce_flat, aE_flat, bE_flat, cw_flat, aW_flat, bW_flat):
    k = pl.kernel(
        _attn_kernel,
        out_type=(jax.ShapeDtypeStruct((NC, B, D), jnp.float32),
                  jax.ShapeDtypeStruct((NC, B, D), jnp.float32)),
        mesh=_MESH,
        scratch_types=[
            pltpu.VMEM_SHARED((B, D), jnp.float32),
            pltpu.VMEM_SHARED((B, D), jnp.float32),
            pltpu.VMEM((CHUNK, D), jnp.float32),
            pltpu.VMEM((CHUNK, D), jnp.float32),
            pltpu.VMEM((1, CHUNK), jnp.int32),
            pltpu.VMEM((1, CHUNK), jnp.int32),
            pltpu.VMEM((1, CHUNK), jnp.int32),
            pltpu.VMEM((1, CHUNK), jnp.int32),
            pltpu.VMEM((CHUNK,), jnp.float32),
            pltpu.VMEM((CHUNK,), jnp.float32),
            pltpu.SemaphoreType.DMA,
            pltpu.SemaphoreType.DMA,
            pltpu.SemaphoreType.DMA,
            pltpu.SemaphoreType.DMA,
            pltpu.SemaphoreType.DMA,
        ],
        compiler_params=_SC_PARAMS,
    )
    accE2, accW2 = k(E, wrep, ce_flat, aE_flat, bE_flat,
                     cw_flat, aW_flat, bW_flat)
    return accE2[0] + accE2[1], accW2[0] + accW2[1]


ETILE = 2048


def _combine_kernel(comp_ref, basis_ref, out_ref):
    b = basis_ref[...].reshape(N_BASES, ETILE * D)
    out_ref[...] = jnp.dot(comp_ref[...], b,
                           preferred_element_type=jnp.float32).reshape(
                               N_REL, ETILE, D)


def _tc_combine(rgcn_comp, rgcn_basis):
    out = pl.pallas_call(
        _combine_kernel,
        grid=(pl.cdiv(N_ENTITY, ETILE),),
        in_specs=[
            pl.BlockSpec((N_REL, N_BASES), lambda i: (0, 0)),
            pl.BlockSpec((N_BASES, ETILE, D), lambda i: (0, i, 0)),
        ],
        out_specs=pl.BlockSpec((N_REL, ETILE, D), lambda i: (0, i, 0)),
        out_shape=jax.ShapeDtypeStruct((N_REL, N_ENTITY, D), jnp.float32),
    )(rgcn_comp, rgcn_basis)
    return out.reshape(N_REL * N_ENTITY, D)


def _y_kernel(we_ref, gw_ref, dinv_ref, out_ref):
    xw = jnp.dot(we_ref[...], gw_ref[...], preferred_element_type=jnp.float32)
    out_ref[...] = dinv_ref[...][:, None] * xw


def _tc_y(word_embedding, gcn_weight, dinv):
    return pl.pallas_call(
        _y_kernel,
        grid=(pl.cdiv(N_WORDS, ETILE),),
        in_specs=[
            pl.BlockSpec((ETILE, D), lambda i: (i, 0)),
            pl.BlockSpec((D, D), lambda i: (0, 0)),
            pl.BlockSpec((ETILE,), lambda i: (i,)),
        ],
        out_specs=pl.BlockSpec((ETILE, D), lambda i: (i, 0)),
        out_shape=jax.ShapeDtypeStruct((N_WORDS, D), jnp.float32),
    )(word_embedding, gcn_weight, dinv)


def _tvec_kernel(x_ref, wa_ref, b_ref, out_ref):
    t = jnp.tanh(jnp.dot(x_ref[...], wa_ref[...],
                         preferred_element_type=jnp.float32))
    out_ref[...] = jnp.dot(t, b_ref[...], preferred_element_type=jnp.float32)


def _tc_tvec(x, Wa, b):
    n = x.shape[0]
    return pl.pallas_call(
        _tvec_kernel,
        grid=(pl.cdiv(n, ETILE),),
        in_specs=[
            pl.BlockSpec((ETILE, D), lambda i: (i, 0)),
            pl.BlockSpec((D, D), lambda i: (0, 0)),
            pl.BlockSpec((D,), lambda i: (0,)),
        ],
        out_specs=pl.BlockSpec((ETILE,), lambda i: (i,)),
        out_shape=jax.ShapeDtypeStruct((n,), jnp.float32),
    )(x, Wa, b)


def _softmax_kernel(s_ref, idx_ref, out_ref):
    s = jnp.where(idx_ref[...] != 0, s_ref[...], -1e30)
    m = jnp.max(s, axis=-1, keepdims=True)
    e = jnp.exp(s - m)
    out_ref[...] = e / jnp.sum(e, axis=-1, keepdims=True)


def _tc_softmax(scores_flat, idx, L):
    return pl.pallas_call(
        _softmax_kernel,
        out_shape=jax.ShapeDtypeStruct((B, L), jnp.float32),
    )(scores_flat.reshape(B, L), idx.astype(jnp.int32)).reshape(-1)


def _gate_kernel(kg_ref, wa_ref, w1_ref, w2_ref, b_ref, out_ref):
    kg = kg_ref[...]
    wa = wa_ref[...]
    z = (jnp.dot(kg, w1_ref[...], preferred_element_type=jnp.float32)
         + jnp.dot(wa, w2_ref[...], preferred_element_type=jnp.float32)
         + b_ref[...][None, :])
    g = jax.nn.sigmoid(z)
    out_ref[...] = g * kg + (1.0 - g) * wa


def _tc_gate(kg, wa, gate_W, gate_b):
    return pl.pallas_call(
        _gate_kernel,
        out_shape=jax.ShapeDtypeStruct((B, D), jnp.float32),
    )(kg, wa, gate_W[:D], gate_W[D:], gate_b)


def _sims_kernel(u_ref, e_ref, bias_ref, out_ref):
    out_ref[...] = lax.dot_general(
        u_ref[...], e_ref[...], (((1,), (1,)), ((), ())),
        preferred_element_type=jnp.float32) + bias_ref[...][None, :]


def _tc_sims(u, E, rec_bias):
    return pl.pallas_call(
        _sims_kernel,
        grid=(pl.cdiv(N_ENTITY, ETILE),),
        in_specs=[
            pl.BlockSpec((B, D), lambda i: (0, 0)),
            pl.BlockSpec((ETILE, D), lambda i: (i, 0)),
            pl.BlockSpec((ETILE,), lambda i: (i,)),
        ],
        out_specs=pl.BlockSpec((B, ETILE), lambda i: (0, i)),
        out_shape=jax.ShapeDtypeStruct((B, N_ENTITY), jnp.float32),
    )(u, E, rec_bias)


def kernel(context_entities, context_words, context_tokens,
           llm_compressed_tokens, edge_index, edge_type, word_edge_index,
           rgcn_basis, rgcn_comp, rgcn_root, rgcn_bias,
           word_embedding, gcn_weight, gcn_bias,
           ent_attn_Wa, ent_attn_b, word_attn_Wa, word_attn_b,
           gate_W, gate_b, rec_bias):
    src, dst = edge_index[0], edge_index[1]
    rel = edge_type
    seg = dst * N_REL + rel
    gidx = rel * N_ENTITY + src
    ws, wd = word_edge_index[0], word_edge_index[1]

    cnt, deg = _sc_histograms(seg.astype(jnp.int32), wd.astype(jnp.int32))
    deg = deg + 1.0

    weight = _tc_combine(rgcn_comp, rgcn_basis)
    winv = 1.0 / jnp.maximum(cnt, 1.0)
    accE = _sc_ent_edges(weight, winv, gidx.astype(jnp.int32),
                         seg.astype(jnp.int32), dst.astype(jnp.int32))
    E = accE + rgcn_root + rgcn_bias

    dinv = 1.0 / jnp.sqrt(jnp.maximum(deg, 1.0))
    y = _tc_y(word_embedding, gcn_weight, dinv)
    accW = _sc_word_edges(y, ws.astype(jnp.int32), wd.astype(jnp.int32))
    wrep = dinv[:, None] * (accW + y) + gcn_bias

    tvec_e = _tc_tvec(E, ent_attn_Wa, ent_attn_b)
    tvec_w = _tc_tvec(wrep, word_attn_Wa, word_attn_b)
    ce_flat = context_entities.reshape(-1).astype(jnp.int32)
    cw_flat = context_words.reshape(-1).astype(jnp.int32)
    sE, sW = _sc_scores(tvec_e, tvec_w, ce_flat, cw_flat)

    aE = _tc_softmax(sE, context_entities, LE)
    aW = _tc_softmax(sW, context_words, LW)
    bE_flat = jnp.repeat(jnp.arange(B, dtype=jnp.int32), LE)
    bW_flat = jnp.repeat(jnp.arange(B, dtype=jnp.int32), LW)
    kg, wa = _sc_attn(E, wrep, ce_flat, aE, bE_flat, cw_flat, aW, bW_flat)
    u = _tc_gate(kg, wa, gate_W, gate_b)
    return _tc_sims(u, E, rec_bias)


# speedup vs baseline: 7.8098x; 1.0311x over previous
import dataclasses
import functools

import jax
import jax.numpy as jnp
from jax import lax
from jax.experimental import pallas as pl
from jax.experimental.pallas import tpu as pltpu
from jax.experimental.pallas import tpu_sc as plsc

N_ENTITY = 10000
N_WORDS = 20000
N_REL = 12
N_BASES = 8
D = 128
B = 1024
LE = 50
LW = 200
E_ENT = 320000
E_WORD = 320000

NC = 2
NS = 16
NW = NC * NS

_MESH = plsc.VectorSubcoreMesh(core_axis_name="c", subcore_axis_name="s")

_SC_PARAMS = pltpu.CompilerParams()
if "needs_layout_passes" in pltpu.CompilerParams.__dataclass_fields__:
    _SC_PARAMS = dataclasses.replace(_SC_PARAMS, needs_layout_passes=False)

CNT_Q = 7552
CNT_PAD = NS * CNT_Q
DEG_Q = 1280
DEG_PAD = NS * DEG_Q

CHUNK = 128


def _fill16(ref, val, n):
    @pl.loop(0, n, step=16)
    def _(i):
        ref[pl.ds(i, 16)] = jnp.full((16,), val, ref.dtype)


def _hist_kernel(seg_hbm, dstw_hbm, cnt_out, deg_out,
                 cnt_sp, deg_sp, zbuf, ones_v, idxA, idxB,
                 semA, semB, semS):
    c = lax.axis_index("c")
    s = lax.axis_index("s")
    wid = c * NS + s

    _fill16(zbuf, 0.0, CNT_Q)
    pltpu.sync_copy(zbuf.at[pl.ds(0, CNT_Q)], cnt_sp.at[pl.ds(s * CNT_Q, CNT_Q)])
    pltpu.sync_copy(zbuf.at[pl.ds(0, DEG_Q)], deg_sp.at[pl.ds(s * DEG_Q, DEG_Q)])
    _fill16(ones_v, 1.0, CHUNK)
    plsc.subcore_barrier()

    def hist_pass(idx_hbm, acc_sp, n_chunks):
        main = n_chunks - n_chunks % (2 * NW)

        @pl.loop(wid, main, step=2 * NW)
        def _(j):
            gA = pltpu.async_copy(idx_hbm.at[pl.ds(j * CHUNK, CHUNK)],
                                  idxA.at[0], semA)
            gB = pltpu.async_copy(idx_hbm.at[pl.ds((j + NW) * CHUNK, CHUNK)],
                                  idxB.at[0], semB)
            gA.wait()
            sA = pltpu.async_copy(ones_v, acc_sp.at[idxA.at[0]], semS,
                                  add=True)
            gB.wait()
            sB = pltpu.async_copy(ones_v, acc_sp.at[idxB.at[0]], semS,
                                  add=True)
            sA.wait()
            sB.wait()

        @pl.when(wid < n_chunks - main)
        def _():
            base = (main + wid) * CHUNK
            pltpu.sync_copy(idx_hbm.at[pl.ds(base, CHUNK)], idxA.at[0])
            pltpu.sync_copy(ones_v, acc_sp.at[idxA.at[0]], add=True)

    hist_pass(seg_hbm, cnt_sp, E_ENT // CHUNK)
    hist_pass(dstw_hbm, deg_sp, E_WORD // CHUNK)

    plsc.subcore_barrier()

    pltpu.sync_copy(cnt_sp.at[pl.ds(s * CNT_Q, CNT_Q)],
                    cnt_out.at[c, pl.ds(s * CNT_Q, CNT_Q)])
    pltpu.sync_copy(deg_sp.at[pl.ds(s * DEG_Q, DEG_Q)],
                    deg_out.at[c, pl.ds(s * DEG_Q, DEG_Q)])


def _sc_histograms(seg_ent, dst_word):
    k = pl.kernel(
        _hist_kernel,
        out_type=(jax.ShapeDtypeStruct((NC, CNT_PAD), jnp.float32),
                  jax.ShapeDtypeStruct((NC, DEG_PAD), jnp.float32)),
        mesh=_MESH,
        scratch_types=[
            pltpu.VMEM_SHARED((CNT_PAD,), jnp.float32),
            pltpu.VMEM_SHARED((DEG_PAD,), jnp.float32),
            pltpu.VMEM((CNT_Q,), jnp.float32),
            pltpu.VMEM((CHUNK,), jnp.float32),
            pltpu.VMEM((1, CHUNK), jnp.int32),
            pltpu.VMEM((1, CHUNK), jnp.int32),
            pltpu.SemaphoreType.DMA,
            pltpu.SemaphoreType.DMA,
            pltpu.SemaphoreType.DMA,
        ],
        compiler_params=_SC_PARAMS,
    )
    cnt2, deg2 = k(seg_ent, dst_word)
    cnt = (cnt2[0] + cnt2[1])[:N_ENTITY * N_REL]
    deg = (deg2[0] + deg2[1])[:N_WORDS]
    return cnt, deg


ACC_E_Q = 640
ACC_E_PAD = NS * ACC_E_Q
ACC_W_Q = 640
ACC_W_PAD = NS * ACC_W_Q


def _zero_shared_rows(acc_sp, rows, s, quota):
    @pl.loop(0, CHUNK)
    def _(r):
        @pl.loop(0, D, step=16)
        def _(k):
            rows[r, pl.ds(k, 16)] = jnp.zeros((16,), jnp.float32)

    @pl.loop(0, quota, step=CHUNK)
    def _(r):
        pltpu.sync_copy(rows, acc_sp.at[pl.ds(s * quota + r, CHUNK)])


def _scale_rows(rows, w_v):
    @pl.loop(0, CHUNK)
    def _(e):
        wsplat = plsc.load_gather(w_v, [jnp.full((16,), e, jnp.int32)])
        for k in range(D // 16):
            sl = rows.at[e, pl.ds(k * 16, 16)]
            sl[...] = sl[...] * wsplat


NBUF = 4


NBUF_E = 2


def _ent_edge_kernel(wflat_hbm, winv_hbm, gidx_hbm, seg_hbm, dst_hbm, acc_out,
                     acc_sp, rows0, rows1, w0, w1,
                     gidxb, segb, dstb,
                     semI0, semI1, semG0, semG1, semS):
    c = lax.axis_index("c")
    s = lax.axis_index("s")
    wid = c * NS + s
    rows_l = [rows0, rows1]
    w_l = [w0, w1]
    semI = [semI0, semI1]
    semG = [semG0, semG1]

    _zero_shared_rows(acc_sp, rows0, s, ACC_E_Q)
    plsc.subcore_barrier()

    n_chunks = E_ENT // CHUNK
    main = n_chunks - n_chunks % (NBUF_E * NW)

    def idx_load(j, i):
        base = j * CHUNK
        return (
            pltpu.async_copy(gidx_hbm.at[pl.ds(base, CHUNK)], gidxb.at[i],
                             semI[i]),
            pltpu.async_copy(seg_hbm.at[pl.ds(base, CHUNK)], segb.at[i],
                             semI[i]),
            pltpu.async_copy(dst_hbm.at[pl.ds(base, CHUNK)], dstb.at[i],
                             semI[i]),
        )

    @pl.loop(wid, main, step=NBUF_E * NW)
    def _(j):
        hI = [idx_load(j + i * NW, i) for i in range(NBUF_E)]
        hG = []
        for i in range(NBUF_E):
            for h in hI[i]:
                h.wait()
            hG.append((
                pltpu.async_copy(wflat_hbm.at[gidxb.at[i]], rows_l[i],
                                 semG[i]),
                pltpu.async_copy(winv_hbm.at[segb.at[i]], w_l[i], semG[i]),
            ))
        hS = []
        for i in range(NBUF_E):
            for h in hG[i]:
                h.wait()
            _scale_rows(rows_l[i], w_l[i])
            hS.append(pltpu.async_copy(rows_l[i], acc_sp.at[dstb.at[i]],
                                       semS, add=True))
        for h in hS:
            h.wait()

    @pl.loop(main + wid, n_chunks, step=NW)
    def _(j):
        for h in idx_load(j, 0):
            h.wait()
        g_r = pltpu.async_copy(wflat_hbm.at[gidxb.at[0]], rows0, semG0)
        g_w = pltpu.async_copy(winv_hbm.at[segb.at[0]], w0, semG0)
        g_w.wait()
        g_r.wait()
        _scale_rows(rows0, w0)
        pltpu.sync_copy(rows0, acc_sp.at[dstb.at[0]], add=True)

    plsc.subcore_barrier()

    @pl.loop(0, ACC_E_Q, step=CHUNK)
    def _(r):
        pltpu.sync_copy(acc_sp.at[pl.ds(s * ACC_E_Q + r, CHUNK)],
                        acc_out.at[c, pl.ds(s * ACC_E_Q + r, CHUNK)])


def _word_edge_kernel(y_hbm, src_hbm, dst_hbm, acc_out,
                      acc_sp, rowsA, rowsB, srcA, srcB, dlA, dlB,
                      lidxA, lidxB, semA, semB, semS):
    c = lax.axis_index("c")
    s = lax.axis_index("s")

    _zero_shared_rows(acc_sp, rowsA, s, ACC_W_Q)
    plsc.subcore_barrier()

    n_chunks = E_WORD // CHUNK
    main = n_chunks - n_chunks % (2 * NS)
    lo = c * N_ENTITY

    def fixup(dl_v, lidx_v):
        for k in range(CHUNK // 16):
            v = dl_v[pl.ds(k * 16, 16)] - lo
            inb = (v >= 0) & (v < 10000)
            trash = jnp.full((16,), 10000 + k * 16, jnp.int32) + lax.iota(
                jnp.int32, 16)
            lidx_v[pl.ds(k * 16, 16)] = jnp.where(inb, v, trash)

    def load_and_gather(j, src_v, dl_v, rows, sem_r):
        base = j * CHUNK
        pltpu.sync_copy(src_hbm.at[pl.ds(base, CHUNK)], src_v.at[0])
        pltpu.sync_copy(dst_hbm.at[pl.ds(base, CHUNK)], dl_v)
        return pltpu.async_copy(y_hbm.at[src_v.at[0]], rows, sem_r)

    @pl.loop(s, main, step=2 * NS)
    def _(j):
        gA = load_and_gather(j, srcA, dlA, rowsA, semA)
        gB = load_and_gather(j + NS, srcB, dlB, rowsB, semB)
        fixup(dlA, lidxA)
        fixup(dlB, lidxB)
        gA.wait()
        sA = pltpu.async_copy(rowsA, acc_sp.at[lidxA], semS, add=True)
        gB.wait()
        sB = pltpu.async_copy(rowsB, acc_sp.at[lidxB], semS, add=True)
        sA.wait()
        sB.wait()

    @pl.when(s < n_chunks - main)
    def _():
        gA = load_and_gather(main + s, srcA, dlA, rowsA, semA)
        fixup(dlA, lidxA)
        gA.wait()
        pltpu.sync_copy(rowsA, acc_sp.at[lidxA], add=True)

    plsc.subcore_barrier()

    @pl.loop(0, ACC_W_Q, step=CHUNK)
    def _(r):
        pltpu.sync_copy(acc_sp.at[pl.ds(s * ACC_W_Q + r, CHUNK)],
                        acc_out.at[c, pl.ds(s * ACC_W_Q + r, CHUNK)])


def _sc_ent_edges(wflat, winv, gidx, seg, dst):
    k = pl.kernel(
        _ent_edge_kernel,
        out_type=jax.ShapeDtypeStruct((NC, ACC_E_PAD, D), jnp.float32),
        mesh=_MESH,
        scratch_types=(
            [pltpu.VMEM_SHARED((ACC_E_PAD, D), jnp.float32)]
            + [pltpu.VMEM((CHUNK, D), jnp.float32)] * NBUF_E
            + [pltpu.VMEM((CHUNK,), jnp.float32)] * NBUF_E
            + [pltpu.VMEM((NBUF_E, CHUNK), jnp.int32)] * 3
            + [pltpu.SemaphoreType.DMA] * (2 * NBUF_E + 1)
        ),
        compiler_params=_SC_PARAMS,
    )
    acc2 = k(wflat, winv, gidx, seg, dst)
    return (acc2[0] + acc2[1])[:N_ENTITY]


def _sc_word_edges(y, src_w, dst_w):
    k = pl.kernel(
        _word_edge_kernel,
        out_type=jax.ShapeDtypeStruct((NC, ACC_W_PAD, D), jnp.float32),
        mesh=_MESH,
        scratch_types=[
            pltpu.VMEM_SHARED((ACC_W_PAD, D), jnp.float32),
            pltpu.VMEM((CHUNK, D), jnp.float32),
            pltpu.VMEM((CHUNK, D), jnp.float32),
            pltpu.VMEM((1, CHUNK), jnp.int32),
            pltpu.VMEM((1, CHUNK), jnp.int32),
            pltpu.VMEM((CHUNK,), jnp.int32),
            pltpu.VMEM((CHUNK,), jnp.int32),
            pltpu.VMEM((CHUNK,), jnp.int32),
            pltpu.VMEM((CHUNK,), jnp.int32),
            pltpu.SemaphoreType.DMA,
            pltpu.SemaphoreType.DMA,
            pltpu.SemaphoreType.DMA,
        ],
        compiler_params=_SC_PARAMS,
    )
    acc2 = k(y, src_w, dst_w)
    return jnp.concatenate([acc2[0, :N_ENTITY], acc2[1, :N_ENTITY]], axis=0)


TVE_PAD = 10240
TVW_PAD = 20480
SE_N = B * LE
SW_N = B * LW


def _scores_kernel(tve_hbm, tvw_hbm, ce_hbm, cw_hbm, se_out, sw_out,
                   tve_sp, tvw_sp, idx_v, s_v, sem):
    c = lax.axis_index("c")
    s = lax.axis_index("s")
    wid = c * NS + s

    pltpu.sync_copy(tve_hbm.at[pl.ds(s * 640, 640)],
                    tve_sp.at[pl.ds(s * 640, 640)])
    pltpu.sync_copy(tvw_hbm.at[pl.ds(s * 1280, 1280)],
                    tvw_sp.at[pl.ds(s * 1280, 1280)])
    plsc.subcore_barrier()

    @pl.loop(wid, SE_N // CHUNK, step=NW)
    def _(j):
        base = j * CHUNK
        pltpu.sync_copy(ce_hbm.at[pl.ds(base, CHUNK)], idx_v.at[0])
        pltpu.async_copy(tve_sp.at[idx_v.at[0]], s_v, sem).wait()
        pltpu.sync_copy(s_v, se_out.at[pl.ds(base, CHUNK)])

    @pl.loop(wid, SW_N // CHUNK, step=NW)
    def _(j):
        base = j * CHUNK
        pltpu.sync_copy(cw_hbm.at[pl.ds(base, CHUNK)], idx_v.at[0])
        pltpu.async_copy(tvw_sp.at[idx_v.at[0]], s_v, sem).wait()
        pltpu.sync_copy(s_v, sw_out.at[pl.ds(base, CHUNK)])


def _sc_scores(tvec_e, tvec_w, ce_flat, cw_flat):
    k = pl.kernel(
        _scores_kernel,
        out_type=(jax.ShapeDtypeStruct((SE_N,), jnp.float32),
                  jax.ShapeDtypeStruct((SW_N,), jnp.float32)),
        mesh=_MESH,
        scratch_types=[
            pltpu.VMEM_SHARED((TVE_PAD,), jnp.float32),
            pltpu.VMEM_SHARED((TVW_PAD,), jnp.float32),
            pltpu.VMEM((1, CHUNK), jnp.int32),
            pltpu.VMEM((CHUNK,), jnp.float32),
            pltpu.SemaphoreType.DMA,
        ],
        compiler_params=_SC_PARAMS,
    )
    tve = jnp.pad(tvec_e, (0, TVE_PAD - N_ENTITY))
    tvw = jnp.pad(tvec_w, (0, TVW_PAD - N_WORDS))
    return k(tve, tvw, ce_flat, cw_flat)


def _attn_kernel(e_hbm, w_hbm, ce_hbm, ae_hbm, be_hbm, cw_hbm, aw_hbm,
                 bw_hbm, acce_out, accw_out,
                 acce_sp, accw_sp, rows0, rows1, rows2, rows3,
                 a0, a1, a2, a3, idxb, bb,
                 semI0, semI1, semI2, semI3,
                 semG0, semG1, semG2, semG3, semS):
    c = lax.axis_index("c")
    s = lax.axis_index("s")
    wid = c * NS + s
    rows_l = [rows0, rows1, rows2, rows3]
    a_l = [a0, a1, a2, a3]
    semI = [semI0, semI1, semI2, semI3]
    semG = [semG0, semG1, semG2, semG3]

    @pl.loop(0, CHUNK)
    def _(r):
        @pl.loop(0, D, step=16)
        def _(k):
            rows0[r, pl.ds(k, 16)] = jnp.zeros((16,), jnp.float32)

    pltpu.sync_copy(rows0.at[pl.ds(0, 64)], acce_sp.at[pl.ds(s * 64, 64)])
    pltpu.sync_copy(rows0.at[pl.ds(0, 64)], accw_sp.at[pl.ds(s * 64, 64)])
    plsc.subcore_barrier()

    def pass_over(table_hbm, idx_hbm, a_hbm, bidx_hbm, acc_sp, n_pairs):
        n_chunks = n_pairs // CHUNK
        main = n_chunks - n_chunks % (NBUF * NW)

        def idx_load(j, i):
            base = j * CHUNK
            return (
                pltpu.async_copy(idx_hbm.at[pl.ds(base, CHUNK)], idxb.at[i],
                                 semI[i]),
                pltpu.async_copy(bidx_hbm.at[pl.ds(base, CHUNK)], bb.at[i],
                                 semI[i]),
                pltpu.async_copy(a_hbm.at[pl.ds(base, CHUNK)], a_l[i],
                                 semI[i]),
            )

        @pl.loop(wid, main, step=NBUF * NW)
        def _(j):
            hI = [idx_load(j + i * NW, i) for i in range(NBUF)]
            hG = []
            for i in range(NBUF):
                for h in hI[i]:
                    h.wait()
                hG.append(pltpu.async_copy(table_hbm.at[idxb.at[i]],
                                           rows_l[i], semG[i]))
            hS = []
            for i in range(NBUF):
                hG[i].wait()
                _scale_rows(rows_l[i], a_l[i])
                hS.append(pltpu.async_copy(rows_l[i], acc_sp.at[bb.at[i]],
                                           semS, add=True))
            for h in hS:
                h.wait()

        @pl.loop(main + wid, n_chunks, step=NW)
        def _(j):
            for h in idx_load(j, 0):
                h.wait()
            pltpu.async_copy(table_hbm.at[idxb.at[0]], rows0, semG0).wait()
            _scale_rows(rows0, a0)
            pltpu.sync_copy(rows0, acc_sp.at[bb.at[0]], add=True)

    pass_over(e_hbm, ce_hbm, ae_hbm, be_hbm, acce_sp, SE_N)
    pass_over(w_hbm, cw_hbm, aw_hbm, bw_hbm, accw_sp, SW_N)
    plsc.subcore_barrier()

    pltpu.sync_copy(acce_sp.at[pl.ds(s * 64, 64)],
                    acce_out.at[c, pl.ds(s * 64, 64)])
    pltpu.sync_copy(accw_sp.at[pl.ds(s * 64, 64)],
                    accw_out.at[c, pl.ds(s * 64, 64)])


def _sc_attn(E, wrep, ce_flat, aE_flat, bE_flat, cw_flat, aW_flat, bW_flat):
    k = pl.kernel(
        _attn_kernel,
        out_type=(jax.ShapeDtypeStruct((NC, B, D), jnp.float32),
                  jax.ShapeDtypeStruct((NC, B, D), jnp.float32)),
        mesh=_MESH,
        scratch_types=(
            [pltpu.VMEM_SHARED((B, D), jnp.float32)] * 2
            + [pltpu.VMEM((CHUNK, D), jnp.float32)] * NBUF
            + [pltpu.VMEM((CHUNK,), jnp.float32)] * NBUF
            + [pltpu.VMEM((NBUF, CHUNK), jnp.int32)] * 2
            + [pltpu.SemaphoreType.DMA] * (2 * NBUF + 1)
        ),
        compiler_params=_SC_PARAMS,
    )
    accE2, accW2 = k(E, wrep, ce_flat, aE_flat, bE_flat,
                     cw_flat, aW_flat, bW_flat)
    return accE2[0] + accE2[1], accW2[0] + accW2[1]


ETILE = 2048


def _combine_kernel(comp_ref, basis_ref, out_ref):
    b = basis_ref[...].reshape(N_BASES, ETILE * D)
    out_ref[...] = jnp.dot(comp_ref[...], b,
                           preferred_element_type=jnp.float32).reshape(
                               N_REL, ETILE, D)


def _tc_combine(rgcn_comp, rgcn_basis):
    out = pl.pallas_call(
        _combine_kernel,
        grid=(pl.cdiv(N_ENTITY, ETILE),),
        in_specs=[
            pl.BlockSpec((N_REL, N_BASES), lambda i: (0, 0)),
            pl.BlockSpec((N_BASES, ETILE, D), lambda i: (0, i, 0)),
        ],
        out_specs=pl.BlockSpec((N_REL, ETILE, D), lambda i: (0, i, 0)),
        out_shape=jax.ShapeDtypeStruct((N_REL, N_ENTITY, D), jnp.float32),
    )(rgcn_comp, rgcn_basis)
    return out.reshape(N_REL * N_ENTITY, D)


def _y_kernel(we_ref, gw_ref, dinv_ref, out_ref):
    xw = jnp.dot(we_ref[...], gw_ref[...], preferred_element_type=jnp.float32)
    out_ref[...] = dinv_ref[...][:, None] * xw


def _tc_y(word_embedding, gcn_weight, dinv):
    return pl.pallas_call(
        _y_kernel,
        grid=(pl.cdiv(N_WORDS, ETILE),),
        in_specs=[
            pl.BlockSpec((ETILE, D), lambda i: (i, 0)),
            pl.BlockSpec((D, D), lambda i: (0, 0)),
            pl.BlockSpec((ETILE,), lambda i: (i,)),
        ],
        out_specs=pl.BlockSpec((ETILE, D), lambda i: (i, 0)),
        out_shape=jax.ShapeDtypeStruct((N_WORDS, D), jnp.float32),
    )(word_embedding, gcn_weight, dinv)


def _tvec_kernel(x_ref, wa_ref, b_ref, out_ref):
    t = jnp.tanh(jnp.dot(x_ref[...], wa_ref[...],
                         preferred_element_type=jnp.float32))
    out_ref[...] = jnp.dot(t, b_ref[...], preferred_element_type=jnp.float32)


def _tc_tvec(x, Wa, b):
    n = x.shape[0]
    return pl.pallas_call(
        _tvec_kernel,
        grid=(pl.cdiv(n, ETILE),),
        in_specs=[
            pl.BlockSpec((ETILE, D), lambda i: (i, 0)),
            pl.BlockSpec((D, D), lambda i: (0, 0)),
            pl.BlockSpec((D,), lambda i: (0,)),
        ],
        out_specs=pl.BlockSpec((ETILE,), lambda i: (i,)),
        out_shape=jax.ShapeDtypeStruct((n,), jnp.float32),
    )(x, Wa, b)


def _softmax_kernel(s_ref, idx_ref, out_ref):
    s = jnp.where(idx_ref[...] != 0, s_ref[...], -1e30)
    m = jnp.max(s, axis=-1, keepdims=True)
    e = jnp.exp(s - m)
    out_ref[...] = e / jnp.sum(e, axis=-1, keepdims=True)


def _tc_softmax(scores_flat, idx, L):
    return pl.pallas_call(
        _softmax_kernel,
        out_shape=jax.ShapeDtypeStruct((B, L), jnp.float32),
    )(scores_flat.reshape(B, L), idx.astype(jnp.int32)).reshape(-1)


def _gate_kernel(kg_ref, wa_ref, w1_ref, w2_ref, b_ref, out_ref):
    kg = kg_ref[...]
    wa = wa_ref[...]
    z = (jnp.dot(kg, w1_ref[...], preferred_element_type=jnp.float32)
         + jnp.dot(wa, w2_ref[...], preferred_element_type=jnp.float32)
         + b_ref[...][None, :])
    g = jax.nn.sigmoid(z)
    out_ref[...] = g * kg + (1.0 - g) * wa


def _tc_gate(kg, wa, gate_W, gate_b):
    return pl.pallas_call(
        _gate_kernel,
        out_shape=jax.ShapeDtypeStruct((B, D), jnp.float32),
    )(kg, wa, gate_W[:D], gate_W[D:], gate_b)


def _sims_kernel(u_ref, e_ref, bias_ref, out_ref):
    out_ref[...] = lax.dot_general(
        u_ref[...], e_ref[...], (((1,), (1,)), ((), ())),
        preferred_element_type=jnp.float32) + bias_ref[...][None, :]


def _tc_sims(u, E, rec_bias):
    return pl.pallas_call(
        _sims_kernel,
        grid=(pl.cdiv(N_ENTITY, ETILE),),
        in_specs=[
            pl.BlockSpec((B, D), lambda i: (0, 0)),
            pl.BlockSpec((ETILE, D), lambda i: (i, 0)),
            pl.BlockSpec((ETILE,), lambda i: (i,)),
        ],
        out_specs=pl.BlockSpec((B, ETILE), lambda i: (0, i)),
        out_shape=jax.ShapeDtypeStruct((B, N_ENTITY), jnp.float32),
    )(u, E, rec_bias)


def kernel(context_entities, context_words, context_tokens,
           llm_compressed_tokens, edge_index, edge_type, word_edge_index,
           rgcn_basis, rgcn_comp, rgcn_root, rgcn_bias,
           word_embedding, gcn_weight, gcn_bias,
           ent_attn_Wa, ent_attn_b, word_attn_Wa, word_attn_b,
           gate_W, gate_b, rec_bias):
    src, dst = edge_index[0], edge_index[1]
    rel = edge_type
    seg = dst * N_REL + rel
    gidx = rel * N_ENTITY + src
    ws, wd = word_edge_index[0], word_edge_index[1]

    cnt, deg = _sc_histograms(seg.astype(jnp.int32), wd.astype(jnp.int32))
    deg = deg + 1.0

    weight = _tc_combine(rgcn_comp, rgcn_basis)
    winv = 1.0 / jnp.maximum(cnt, 1.0)
    accE = _sc_ent_edges(weight, winv, gidx.astype(jnp.int32),
                         seg.astype(jnp.int32), dst.astype(jnp.int32))
    E = accE + rgcn_root + rgcn_bias

    dinv = 1.0 / jnp.sqrt(jnp.maximum(deg, 1.0))
    y = _tc_y(word_embedding, gcn_weight, dinv)
    accW = _sc_word_edges(y, ws.astype(jnp.int32), wd.astype(jnp.int32))
    wrep = dinv[:, None] * (accW + y) + gcn_bias

    tvec_e = _tc_tvec(E, ent_attn_Wa, ent_attn_b)
    tvec_w = _tc_tvec(wrep, word_attn_Wa, word_attn_b)
    ce_flat = context_entities.reshape(-1).astype(jnp.int32)
    cw_flat = context_words.reshape(-1).astype(jnp.int32)
    sE, sW = _sc_scores(tvec_e, tvec_w, ce_flat, cw_flat)

    aE = _tc_softmax(sE, context_entities, LE)
    aW = _tc_softmax(sW, context_words, LW)
    bE_flat = jnp.repeat(jnp.arange(B, dtype=jnp.int32), LE)
    bW_flat = jnp.repeat(jnp.arange(B, dtype=jnp.int32), LW)
    kg, wa = _sc_attn(E, wrep, ce_flat, aE, bE_flat, cw_flat, aW, bW_flat)
    u = _tc_gate(kg, wa, gate_W, gate_b)
    return _tc_sims(u, E, rec_bias)
